# baseline (speedup 1.0000x reference)
import functools

import jax
import jax.numpy as jnp
from jax import lax
from jax.experimental import pallas as pl
from jax.experimental.pallas import tpu as pltpu
from jax.experimental.pallas import tpu_sc as plsc

N_NODES = 10000
N_EDGES = 320000
D = 128
C = 40

NC = 2
NS = 16
CHUNK = 64

E_PAD = 327680
N_ROWS = E_PAD // CHUNK
SEG_RPW = N_ROWS // NS
DEG_RPW = N_ROWS // (NC * NS)
N_ACC = 10240
ZROWS = N_ACC // NS
NBUF = 4
IDXBLK = 40
N_IDXBLK = SEG_RPW // IDXBLK
SUP_PER_BLK = IDXBLK // NBUF

_mesh = plsc.VectorSubcoreMesh(core_axis_name="c", subcore_axis_name="s")


def _zero_vmem_2d(ref, nrows):
    zv = jnp.zeros((16,), jnp.float32)

    def body(i, _):
        r = i // 8
        col = (i % 8) * 16
        ref[r, pl.ds(col, 16)] = zv
        return 0

    lax.fori_loop(0, nrows * 8, body, 0)


@functools.partial(
    pl.kernel,
    out_type=jax.ShapeDtypeStruct((2, N_ACC), jnp.float32),
    mesh=_mesh,
    scratch_types=[
        pltpu.VMEM_SHARED((N_ACC,), jnp.float32),
        pltpu.VMEM((640,), jnp.float32),
        pltpu.VMEM((CHUNK,), jnp.float32),
        pltpu.VMEM((CHUNK,), jnp.int32),
        pltpu.SemaphoreType.DMA,
    ],
)
def _deg_kernel(dst2_hbm, out_hbm, acc, zbuf, ones_v, idx_v, sem):
    c = lax.axis_index("c")
    s = lax.axis_index("s")

    def fill(i, _):
        zbuf[pl.ds(i * 16, 16)] = jnp.zeros((16,), jnp.float32)
        return 0

    lax.fori_loop(0, 40, fill, 0)

    def fill1(i, _):
        ones_v[pl.ds(i * 16, 16)] = jnp.ones((16,), jnp.float32)
        return 0

    lax.fori_loop(0, CHUNK // 16, fill1, 0)
    pltpu.sync_copy(zbuf, acc.at[pl.ds(s * ZROWS, ZROWS)])
    plsc.subcore_barrier()

    base_row = c * (N_ROWS // 2) + s * DEG_RPW

    def chunk(g, _):
        pltpu.sync_copy(dst2_hbm.at[base_row + g], idx_v)
        pltpu.sync_copy(ones_v, acc.at[idx_v], add=True)
        return 0

    lax.fori_loop(0, DEG_RPW, chunk, 0)
    plsc.subcore_barrier()

    pltpu.sync_copy(acc.at[pl.ds(s * ZROWS, ZROWS)], zbuf)
    pltpu.sync_copy(zbuf, out_hbm.at[c, pl.ds(s * ZROWS, ZROWS)])


@functools.partial(
    pl.kernel,
    out_type=(
        jax.ShapeDtypeStruct((N_ACC, D), jnp.float32),
        jax.ShapeDtypeStruct((N_ACC, D), jnp.float32),
    ),
    mesh=_mesh,
    scratch_types=[
        pltpu.VMEM_SHARED((N_ACC, D), jnp.float32),
        pltpu.VMEM((IDXBLK, CHUNK), jnp.int32),
        pltpu.VMEM((IDXBLK, CHUNK), jnp.int32),
        pltpu.VMEM((NBUF, CHUNK, D), jnp.float32),
        pltpu.SemaphoreType.DMA((NBUF,)),
        pltpu.SemaphoreType.DMA((NBUF,)),
    ],
)
def _segsum_kernel(h_hbm, hs_hbm, src2_hbm, dst2_hbm, s_out, t_out,
                   acc, src_sub, dst_sub, rows_v, gsem, ssem):
    c = lax.axis_index("c")
    s = lax.axis_index("s")

    _zero_vmem_2d(rows_v.at[0], CHUNK)
    for k in range(ZROWS // CHUNK):
        pltpu.sync_copy(rows_v.at[0],
                        acc.at[pl.ds(s * ZROWS + k * CHUNK, CHUNK)])
    plsc.subcore_barrier()

    def fire_gather(b, row):
        @pl.when(c == 0)
        def _():
            pltpu.async_copy(h_hbm.at[src_sub.at[row]], rows_v.at[b],
                             gsem.at[b])

        @pl.when(c == 1)
        def _():
            pltpu.async_copy(hs_hbm.at[src_sub.at[row]], rows_v.at[b],
                             gsem.at[b])

    def wait_gather(b, row):
        pltpu.make_async_copy(h_hbm.at[src_sub.at[row]], rows_v.at[b],
                              gsem.at[b]).wait()

    def fire_scatter(b, row):
        pltpu.async_copy(rows_v.at[b], acc.at[dst_sub.at[row]], ssem.at[b],
                         add=True)

    def wait_scatter(b, row):
        pltpu.make_async_copy(rows_v.at[b], acc.at[dst_sub.at[row]],
                              ssem.at[b]).wait()

    def block_body(blk, _):
        base = s * SEG_RPW + blk * IDXBLK
        pltpu.sync_copy(src2_hbm.at[pl.ds(base, IDXBLK)], src_sub)
        pltpu.sync_copy(dst2_hbm.at[pl.ds(base, IDXBLK)], dst_sub)
        fire_gather(0, 0)
        fire_gather(1, 1)

        def super_body(g, _):
            c0 = g * 4
            wait_gather(0, c0)
            fire_scatter(0, c0)
            wait_gather(1, c0 + 1)
            fire_scatter(1, c0 + 1)

            @pl.when(g > 0)
            def _():
                wait_scatter(2, c0 - 2)
                wait_scatter(3, c0 - 1)

            fire_gather(2, c0 + 2)
            fire_gather(3, c0 + 3)

            wait_gather(2, c0 + 2)
            fire_scatter(2, c0 + 2)
            wait_gather(3, c0 + 3)
            fire_scatter(3, c0 + 3)

            @pl.when(g < SUP_PER_BLK - 1)
            def _():
                wait_scatter(0, c0)
                wait_scatter(1, c0 + 1)
                fire_gather(0, c0 + 4)
                fire_gather(1, c0 + 5)

            return 0

        lax.fori_loop(0, SUP_PER_BLK, super_body, 0)
        last = (SUP_PER_BLK - 1) * 4
        wait_scatter(0, last)
        wait_scatter(1, last + 1)
        wait_scatter(2, last + 2)
        wait_scatter(3, last + 3)
        return 0

    lax.fori_loop(0, N_IDXBLK, block_body, 0)
    plsc.subcore_barrier()

    sl = pl.ds(s * ZROWS, ZROWS)

    @pl.when(c == 0)
    def _():
        pltpu.sync_copy(acc.at[sl], s_out.at[sl])

    @pl.when(c == 1)
    def _():
        pltpu.sync_copy(acc.at[sl], t_out.at[sl])


BN = 1000
_GRID = N_NODES // BN
_HP = jax.lax.Precision.HIGHEST


def _prep_body(x_ref, w_ref, b_ref, d0_ref, d1_ref, h_ref, hs_ref, rsd_ref,
               inv_ref):
    xb = x_ref[...]
    h = jnp.dot(xb, w_ref[...], preferred_element_type=jnp.float32,
                precision=_HP) + b_ref[...]
    deg = jnp.maximum(d0_ref[...] + d1_ref[...], 1.0)
    rsd = lax.rsqrt(deg)
    h_ref[...] = h
    hs_ref[...] = h * rsd
    rsd_ref[...] = rsd
    inv_ref[...] = 1.0 / deg


def _prep_call(x, lin1_w, lin1_b, d0, d1):
    return pl.pallas_call(
        _prep_body,
        grid=(_GRID,),
        in_specs=[
            pl.BlockSpec((BN, D), lambda i: (i, 0)),
            pl.BlockSpec((D, D), lambda i: (0, 0)),
            pl.BlockSpec((1, D), lambda i: (0, 0)),
            pl.BlockSpec((BN, 1), lambda i: (i, 0)),
            pl.BlockSpec((BN, 1), lambda i: (i, 0)),
        ],
        out_specs=[
            pl.BlockSpec((BN, D), lambda i: (i, 0)),
            pl.BlockSpec((BN, D), lambda i: (i, 0)),
            pl.BlockSpec((BN, 1), lambda i: (i, 0)),
            pl.BlockSpec((BN, 1), lambda i: (i, 0)),
        ],
        out_shape=[
            jax.ShapeDtypeStruct((N_NODES, D), jnp.float32),
            jax.ShapeDtypeStruct((N_NODES, D), jnp.float32),
            jax.ShapeDtypeStruct((N_NODES, 1), jnp.float32),
            jax.ShapeDtypeStruct((N_NODES, 1), jnp.float32),
        ],
    )(x, lin1_w, lin1_b.reshape(1, D), d0, d1)


def _elu(v):
    return jnp.where(v > 0, v, jnp.exp(jnp.minimum(v, 0.0)) - 1.0)


def _layer_body(h_ref, s_ref, t_ref, rsd_ref, inv_ref, gw_ref, gb_ref,
                sws_ref, swn_ref, sb_ref, ginw_ref, ginb_ref, naw_ref,
                hn_ref, hns_ref):
    h = h_ref[...]
    S = s_ref[...]
    T = t_ref[...]
    rsd = rsd_ref[...]
    inv = inv_ref[...]
    gcn = jnp.dot(T * rsd, gw_ref[...], preferred_element_type=jnp.float32,
                  precision=_HP) + gb_ref[...]
    sage = (jnp.dot(h, sws_ref[...], preferred_element_type=jnp.float32,
                    precision=_HP)
            + jnp.dot(S * inv, swn_ref[...], preferred_element_type=jnp.float32,
                      precision=_HP) + sb_ref[...])
    gin = jnp.dot(h + S, ginw_ref[...], preferred_element_type=jnp.float32,
                  precision=_HP) + ginb_ref[...]
    w0 = naw_ref[0]
    w1 = naw_ref[1]
    w2 = naw_ref[2]
    o = w0 * _elu(gcn) + w1 * _elu(sage) + w2 * _elu(gin)
    hn_ref[...] = o
    hns_ref[...] = o * rsd


def _layer_call(h, S, T, rsd, inv, gw, gb, sws, swn, sb, ginw, ginb, naw):
    full = lambda i: (0, 0)
    blk = lambda i: (i, 0)
    return pl.pallas_call(
        _layer_body,
        grid=(_GRID,),
        in_specs=[
            pl.BlockSpec((BN, D), blk),
            pl.BlockSpec((BN, D), blk),
            pl.BlockSpec((BN, D), blk),
            pl.BlockSpec((BN, 1), blk),
            pl.BlockSpec((BN, 1), blk),
            pl.BlockSpec((D, D), full),
            pl.BlockSpec((1, D), full),
            pl.BlockSpec((D, D), full),
            pl.BlockSpec((D, D), full),
            pl.BlockSpec((1, D), full),
            pl.BlockSpec((D, D), full),
            pl.BlockSpec((1, D), full),
            pl.BlockSpec(memory_space=pltpu.SMEM),
        ],
        out_specs=[
            pl.BlockSpec((BN, D), blk),
            pl.BlockSpec((BN, D), blk),
        ],
        out_shape=[
            jax.ShapeDtypeStruct((N_NODES, D), jnp.float32),
            jax.ShapeDtypeStruct((N_NODES, D), jnp.float32),
        ],
    )(h, S, T, rsd, inv, gw, gb.reshape(1, D), sws, swn, sb.reshape(1, D),
      ginw, ginb.reshape(1, D), naw)


def _final_body(x1_ref, x2_ref, x3_ref, lcw_ref, lcb_ref, ncw_ref, ncb_ref,
                sc_ref, law_ref, out_ref):
    c1 = sc_ref[0]
    c2 = sc_ref[1]
    a = x3_ref[...]
    b = c1 * x1_ref[...]
    c = c2 * x2_ref[...]
    l_max = jnp.maximum(jnp.maximum(a, b), c)
    l_mean = (a + b + c) * (1.0 / 3.0)
    lcw = lcw_ref[...]
    l_cat = (jnp.dot(a, lcw[0:D], preferred_element_type=jnp.float32,
                     precision=_HP)
             + jnp.dot(b, lcw[D:2 * D], preferred_element_type=jnp.float32,
                       precision=_HP)
             + jnp.dot(c, lcw[2 * D:3 * D], preferred_element_type=jnp.float32,
                       precision=_HP) + lcb_ref[...])
    la0 = law_ref[0]
    la1 = law_ref[1]
    la2 = law_ref[2]
    relu = lambda v: jnp.maximum(v, 0.0)
    x5 = la0 * relu(l_max) + la1 * relu(l_cat) + la2 * relu(l_mean)
    out_ref[...] = jnp.dot(x5, ncw_ref[...], preferred_element_type=jnp.float32,
                           precision=_HP) + ncb_ref[...]


def _final_call(x1, x2, x3, lc_w, lc_b, nc_w, nc_b, sc_c, la_w):
    full = lambda i: (0, 0)
    blk = lambda i: (i, 0)
    return pl.pallas_call(
        _final_body,
        grid=(_GRID,),
        in_specs=[
            pl.BlockSpec((BN, D), blk),
            pl.BlockSpec((BN, D), blk),
            pl.BlockSpec((BN, D), blk),
            pl.BlockSpec((3 * D, D), full),
            pl.BlockSpec((1, D), full),
            pl.BlockSpec((D, C), full),
            pl.BlockSpec((1, C), full),
            pl.BlockSpec(memory_space=pltpu.SMEM),
            pl.BlockSpec(memory_space=pltpu.SMEM),
        ],
        out_specs=pl.BlockSpec((BN, C), blk),
        out_shape=jax.ShapeDtypeStruct((N_NODES, C), jnp.float32),
    )(x1, x2, x3, lc_w, lc_b.reshape(1, D), nc_w, nc_b.reshape(1, C), sc_c,
      la_w)


def kernel(x, edge_index, lin1_w, lin1_b, gcn_w, gcn_b, sage_ws, sage_wn,
           sage_b, gin_w, gin_b, lc_w, lc_b, nc_w, nc_b, na_alphas, sc_alphas,
           la_alphas):
    na_w = jax.nn.softmax(na_alphas, axis=-1)
    sc_w = jax.nn.softmax(sc_alphas, axis=-1)
    la_w = jax.nn.softmax(la_alphas, axis=-1)

    src = edge_index[0]
    dst = edge_index[1]
    pad = E_PAD - N_EDGES
    iota = lax.iota(jnp.int32, pad)
    src_p = jnp.concatenate([src, iota % N_NODES]).reshape(N_ROWS, CHUNK)
    dst_p = jnp.concatenate([dst, N_NODES + (iota % 8)]).reshape(N_ROWS, CHUNK)

    deg_pair = _deg_kernel(dst_p)
    d0 = deg_pair[0, :N_NODES].reshape(N_NODES, 1)
    d1 = deg_pair[1, :N_NODES].reshape(N_NODES, 1)
    h, hs, rsd, inv = _prep_call(x, lin1_w, lin1_b, d0, d1)

    S, T = _segsum_kernel(h, hs, src_p, dst_p)
    x1, x1s = _layer_call(h, S[:N_NODES], T[:N_NODES], rsd, inv, gcn_w[0],
                          gcn_b[0], sage_ws[0], sage_wn[0], sage_b[0],
                          gin_w[0], gin_b[0], na_w[0])
    S, T = _segsum_kernel(x1, x1s, src_p, dst_p)
    x2, x2s = _layer_call(x1, S[:N_NODES], T[:N_NODES], rsd, inv, gcn_w[1],
                          gcn_b[1], sage_ws[1], sage_wn[1], sage_b[1],
                          gin_w[1], gin_b[1], na_w[1])
    S, T = _segsum_kernel(x2, x2s, src_p, dst_p)
    x3, _ = _layer_call(x2, S[:N_NODES], T[:N_NODES], rsd, inv, gcn_w[2],
                        gcn_b[2], sage_ws[2], sage_wn[2], sage_b[2],
                        gin_w[2], gin_b[2], na_w[2])

    sc_c = jnp.stack([sc_w[0, 1], sc_w[1, 1]])
    return _final_call(x1, x2, x3, lc_w, lc_b, nc_w, nc_b, sc_c, la_w[0])


# speedup vs baseline: 1.0636x; 1.0636x over previous
import functools

import jax
import jax.numpy as jnp
from jax import lax
from jax.experimental import pallas as pl
from jax.experimental.pallas import tpu as pltpu
from jax.experimental.pallas import tpu_sc as plsc

N_NODES = 10000
N_EDGES = 320000
D = 128
C = 40

NC = 2
NS = 16
CHUNK = 64

E_PAD = 327680
N_ROWS = E_PAD // CHUNK
SEG_RPW = N_ROWS // NS
DEG_RPW = N_ROWS // (NC * NS)
N_ACC = 10240
ZROWS = N_ACC // NS
NBUF = 4
IDXBLK = 40
N_IDXBLK = SEG_RPW // IDXBLK
SUP_PER_BLK = IDXBLK // NBUF

_mesh = plsc.VectorSubcoreMesh(core_axis_name="c", subcore_axis_name="s")


def _zero_vmem_2d(ref, nrows):
    zv = jnp.zeros((16,), jnp.float32)

    def body(i, _):
        r = i // 8
        col = (i % 8) * 16
        ref[r, pl.ds(col, 16)] = zv
        return 0

    lax.fori_loop(0, nrows * 8, body, 0)


@functools.partial(
    pl.kernel,
    out_type=jax.ShapeDtypeStruct((2, N_ACC), jnp.float32),
    mesh=_mesh,
    scratch_types=[
        pltpu.VMEM_SHARED((N_ACC,), jnp.float32),
        pltpu.VMEM((640,), jnp.float32),
        pltpu.VMEM((CHUNK,), jnp.float32),
        pltpu.VMEM((CHUNK,), jnp.int32),
        pltpu.SemaphoreType.DMA,
    ],
)
def _deg_kernel(dst2_hbm, out_hbm, acc, zbuf, ones_v, idx_v, sem):
    c = lax.axis_index("c")
    s = lax.axis_index("s")

    def fill(i, _):
        zbuf[pl.ds(i * 16, 16)] = jnp.zeros((16,), jnp.float32)
        return 0

    lax.fori_loop(0, 40, fill, 0)

    def fill1(i, _):
        ones_v[pl.ds(i * 16, 16)] = jnp.ones((16,), jnp.float32)
        return 0

    lax.fori_loop(0, CHUNK // 16, fill1, 0)
    pltpu.sync_copy(zbuf, acc.at[pl.ds(s * ZROWS, ZROWS)])
    plsc.subcore_barrier()

    base_row = c * (N_ROWS // 2) + s * DEG_RPW

    def chunk(g, _):
        pltpu.sync_copy(dst2_hbm.at[base_row + g], idx_v)
        pltpu.sync_copy(ones_v, acc.at[idx_v], add=True)
        return 0

    lax.fori_loop(0, DEG_RPW, chunk, 0)
    plsc.subcore_barrier()

    pltpu.sync_copy(acc.at[pl.ds(s * ZROWS, ZROWS)], zbuf)
    pltpu.sync_copy(zbuf, out_hbm.at[c, pl.ds(s * ZROWS, ZROWS)])


@functools.partial(
    pl.kernel,
    out_type=(
        jax.ShapeDtypeStruct((N_ACC, D), jnp.float32),
        jax.ShapeDtypeStruct((N_ACC, D), jnp.float32),
    ),
    mesh=_mesh,
    scratch_types=[
        pltpu.VMEM_SHARED((N_ACC, D), jnp.float32),
        pltpu.VMEM((IDXBLK, CHUNK), jnp.int32),
        pltpu.VMEM((IDXBLK, CHUNK), jnp.int32),
        pltpu.VMEM((NBUF, CHUNK, D), jnp.float32),
        pltpu.SemaphoreType.DMA((NBUF,)),
        pltpu.SemaphoreType.DMA((NBUF,)),
    ],
)
def _segsum_kernel(h_hbm, hs_hbm, src2_hbm, dst2_hbm, s_out, t_out,
                   acc, src_sub, dst_sub, rows_v, gsem, ssem):
    c = lax.axis_index("c")
    s = lax.axis_index("s")

    _zero_vmem_2d(rows_v.at[0], CHUNK)
    for k in range(ZROWS // CHUNK):
        pltpu.sync_copy(rows_v.at[0],
                        acc.at[pl.ds(s * ZROWS + k * CHUNK, CHUNK)])
    plsc.subcore_barrier()

    def fire_gather(b, row):
        @pl.when(c == 0)
        def _():
            pltpu.async_copy(h_hbm.at[src_sub.at[row]], rows_v.at[b],
                             gsem.at[b])

        @pl.when(c == 1)
        def _():
            pltpu.async_copy(hs_hbm.at[src_sub.at[row]], rows_v.at[b],
                             gsem.at[b])

    def wait_gather(b, row):
        pltpu.make_async_copy(h_hbm.at[src_sub.at[row]], rows_v.at[b],
                              gsem.at[b]).wait()

    def fire_scatter(b, row):
        pltpu.async_copy(rows_v.at[b], acc.at[dst_sub.at[row]], ssem.at[b],
                         add=True)

    def wait_scatter(b, row):
        pltpu.make_async_copy(rows_v.at[b], acc.at[dst_sub.at[row]],
                              ssem.at[b]).wait()

    def block_body(blk, _):
        base = s * SEG_RPW + blk * IDXBLK
        pltpu.sync_copy(src2_hbm.at[pl.ds(base, IDXBLK)], src_sub)
        pltpu.sync_copy(dst2_hbm.at[pl.ds(base, IDXBLK)], dst_sub)
        for b in range(NBUF):
            fire_gather(b, b)

        def super_body(g, _):
            for b in range(NBUF):
                wait_gather(b, g * NBUF + b)
                fire_scatter(b, g * NBUF + b)

            @pl.when(g < SUP_PER_BLK - 1)
            def _():
                for b in range(NBUF):
                    wait_scatter(b, g * NBUF + b)
                    fire_gather(b, (g + 1) * NBUF + b)

            return 0

        lax.fori_loop(0, SUP_PER_BLK, super_body, 0)
        for b in range(NBUF):
            wait_scatter(b, (SUP_PER_BLK - 1) * NBUF + b)
        return 0

    lax.fori_loop(0, N_IDXBLK, block_body, 0)
    plsc.subcore_barrier()

    sl = pl.ds(s * ZROWS, ZROWS)

    @pl.when(c == 0)
    def _():
        pltpu.sync_copy(acc.at[sl], s_out.at[sl])

    @pl.when(c == 1)
    def _():
        pltpu.sync_copy(acc.at[sl], t_out.at[sl])


BN = 1000
_GRID = N_NODES // BN
_HP = jax.lax.Precision.HIGHEST


def _prep_body(x_ref, w_ref, b_ref, d0_ref, d1_ref, h_ref, hs_ref, rsd_ref,
               inv_ref):
    xb = x_ref[...]
    h = jnp.dot(xb, w_ref[...], preferred_element_type=jnp.float32,
                precision=_HP) + b_ref[...]
    deg = jnp.maximum(d0_ref[...] + d1_ref[...], 1.0)
    rsd = lax.rsqrt(deg)
    h_ref[...] = h
    hs_ref[...] = h * rsd
    rsd_ref[...] = rsd
    inv_ref[...] = 1.0 / deg


def _prep_call(x, lin1_w, lin1_b, d0, d1):
    return pl.pallas_call(
        _prep_body,
        grid=(_GRID,),
        in_specs=[
            pl.BlockSpec((BN, D), lambda i: (i, 0)),
            pl.BlockSpec((D, D), lambda i: (0, 0)),
            pl.BlockSpec((1, D), lambda i: (0, 0)),
            pl.BlockSpec((BN, 1), lambda i: (i, 0)),
            pl.BlockSpec((BN, 1), lambda i: (i, 0)),
        ],
        out_specs=[
            pl.BlockSpec((BN, D), lambda i: (i, 0)),
            pl.BlockSpec((BN, D), lambda i: (i, 0)),
            pl.BlockSpec((BN, 1), lambda i: (i, 0)),
            pl.BlockSpec((BN, 1), lambda i: (i, 0)),
        ],
        out_shape=[
            jax.ShapeDtypeStruct((N_NODES, D), jnp.float32),
            jax.ShapeDtypeStruct((N_NODES, D), jnp.float32),
            jax.ShapeDtypeStruct((N_NODES, 1), jnp.float32),
            jax.ShapeDtypeStruct((N_NODES, 1), jnp.float32),
        ],
    )(x, lin1_w, lin1_b.reshape(1, D), d0, d1)


def _elu(v):
    return jnp.where(v > 0, v, jnp.exp(jnp.minimum(v, 0.0)) - 1.0)


def _layer_body(h_ref, s_ref, t_ref, rsd_ref, inv_ref, gw_ref, gb_ref,
                sws_ref, swn_ref, sb_ref, ginw_ref, ginb_ref, naw_ref,
                hn_ref, hns_ref):
    h = h_ref[...]
    S = s_ref[...]
    T = t_ref[...]
    rsd = rsd_ref[...]
    inv = inv_ref[...]
    gcn = jnp.dot(T * rsd, gw_ref[...], preferred_element_type=jnp.float32,
                  precision=_HP) + gb_ref[...]
    sage = (jnp.dot(h, sws_ref[...], preferred_element_type=jnp.float32,
                    precision=_HP)
            + jnp.dot(S * inv, swn_ref[...], preferred_element_type=jnp.float32,
                      precision=_HP) + sb_ref[...])
    gin = jnp.dot(h + S, ginw_ref[...], preferred_element_type=jnp.float32,
                  precision=_HP) + ginb_ref[...]
    w0 = naw_ref[0]
    w1 = naw_ref[1]
    w2 = naw_ref[2]
    o = w0 * _elu(gcn) + w1 * _elu(sage) + w2 * _elu(gin)
    hn_ref[...] = o
    hns_ref[...] = o * rsd


def _layer_call(h, S, T, rsd, inv, gw, gb, sws, swn, sb, ginw, ginb, naw):
    full = lambda i: (0, 0)
    blk = lambda i: (i, 0)
    return pl.pallas_call(
        _layer_body,
        grid=(_GRID,),
        in_specs=[
            pl.BlockSpec((BN, D), blk),
            pl.BlockSpec((BN, D), blk),
            pl.BlockSpec((BN, D), blk),
            pl.BlockSpec((BN, 1), blk),
            pl.BlockSpec((BN, 1), blk),
            pl.BlockSpec((D, D), full),
            pl.BlockSpec((1, D), full),
            pl.BlockSpec((D, D), full),
            pl.BlockSpec((D, D), full),
            pl.BlockSpec((1, D), full),
            pl.BlockSpec((D, D), full),
            pl.BlockSpec((1, D), full),
            pl.BlockSpec(memory_space=pltpu.SMEM),
        ],
        out_specs=[
            pl.BlockSpec((BN, D), blk),
            pl.BlockSpec((BN, D), blk),
        ],
        out_shape=[
            jax.ShapeDtypeStruct((N_NODES, D), jnp.float32),
            jax.ShapeDtypeStruct((N_NODES, D), jnp.float32),
        ],
    )(h, S, T, rsd, inv, gw, gb.reshape(1, D), sws, swn, sb.reshape(1, D),
      ginw, ginb.reshape(1, D), naw)


def _final_body(x1_ref, x2_ref, x3_ref, lcw_ref, lcb_ref, ncw_ref, ncb_ref,
                sc_ref, law_ref, out_ref):
    c1 = sc_ref[0]
    c2 = sc_ref[1]
    a = x3_ref[...]
    b = c1 * x1_ref[...]
    c = c2 * x2_ref[...]
    l_max = jnp.maximum(jnp.maximum(a, b), c)
    l_mean = (a + b + c) * (1.0 / 3.0)
    lcw = lcw_ref[...]
    l_cat = (jnp.dot(a, lcw[0:D], preferred_element_type=jnp.float32,
                     precision=_HP)
             + jnp.dot(b, lcw[D:2 * D], preferred_element_type=jnp.float32,
                       precision=_HP)
             + jnp.dot(c, lcw[2 * D:3 * D], preferred_element_type=jnp.float32,
                       precision=_HP) + lcb_ref[...])
    la0 = law_ref[0]
    la1 = law_ref[1]
    la2 = law_ref[2]
    relu = lambda v: jnp.maximum(v, 0.0)
    x5 = la0 * relu(l_max) + la1 * relu(l_cat) + la2 * relu(l_mean)
    out_ref[...] = jnp.dot(x5, ncw_ref[...], preferred_element_type=jnp.float32,
                           precision=_HP) + ncb_ref[...]


def _final_call(x1, x2, x3, lc_w, lc_b, nc_w, nc_b, sc_c, la_w):
    full = lambda i: (0, 0)
    blk = lambda i: (i, 0)
    return pl.pallas_call(
        _final_body,
        grid=(_GRID,),
        in_specs=[
            pl.BlockSpec((BN, D), blk),
            pl.BlockSpec((BN, D), blk),
            pl.BlockSpec((BN, D), blk),
            pl.BlockSpec((3 * D, D), full),
            pl.BlockSpec((1, D), full),
            pl.BlockSpec((D, C), full),
            pl.BlockSpec((1, C), full),
            pl.BlockSpec(memory_space=pltpu.SMEM),
            pl.BlockSpec(memory_space=pltpu.SMEM),
        ],
        out_specs=pl.BlockSpec((BN, C), blk),
        out_shape=jax.ShapeDtypeStruct((N_NODES, C), jnp.float32),
    )(x1, x2, x3, lc_w, lc_b.reshape(1, D), nc_w, nc_b.reshape(1, C), sc_c,
      la_w)


def kernel(x, edge_index, lin1_w, lin1_b, gcn_w, gcn_b, sage_ws, sage_wn,
           sage_b, gin_w, gin_b, lc_w, lc_b, nc_w, nc_b, na_alphas, sc_alphas,
           la_alphas):
    na_w = jax.nn.softmax(na_alphas, axis=-1)
    sc_w = jax.nn.softmax(sc_alphas, axis=-1)
    la_w = jax.nn.softmax(la_alphas, axis=-1)

    src = edge_index[0]
    dst = edge_index[1]
    pad = E_PAD - N_EDGES
    iota = lax.iota(jnp.int32, pad)
    src_p = jnp.concatenate([src, iota % N_NODES]).reshape(N_ROWS, CHUNK)
    dst_p = jnp.concatenate([dst, N_NODES + (iota % 8)]).reshape(N_ROWS, CHUNK)

    deg_pair = _deg_kernel(dst_p)
    d0 = deg_pair[0, :N_NODES].reshape(N_NODES, 1)
    d1 = deg_pair[1, :N_NODES].reshape(N_NODES, 1)
    h, hs, rsd, inv = _prep_call(x, lin1_w, lin1_b, d0, d1)

    S, T = _segsum_kernel(h, hs, src_p, dst_p)
    x1, x1s = _layer_call(h, S[:N_NODES], T[:N_NODES], rsd, inv, gcn_w[0],
                          gcn_b[0], sage_ws[0], sage_wn[0], sage_b[0],
                          gin_w[0], gin_b[0], na_w[0])
    S, T = _segsum_kernel(x1, x1s, src_p, dst_p)
    x2, x2s = _layer_call(x1, S[:N_NODES], T[:N_NODES], rsd, inv, gcn_w[1],
                          gcn_b[1], sage_ws[1], sage_wn[1], sage_b[1],
                          gin_w[1], gin_b[1], na_w[1])
    S, T = _segsum_kernel(x2, x2s, src_p, dst_p)
    x3, _ = _layer_call(x2, S[:N_NODES], T[:N_NODES], rsd, inv, gcn_w[2],
                        gcn_b[2], sage_ws[2], sage_wn[2], sage_b[2],
                        gin_w[2], gin_b[2], na_w[2])

    sc_c = jnp.stack([sc_w[0, 1], sc_w[1, 1]])
    return _final_call(x1, x2, x3, lc_w, lc_b, nc_w, nc_b, sc_c, la_w[0])


# speedup vs baseline: 1.1573x; 1.0881x over previous
import functools

import jax
import jax.numpy as jnp
from jax import lax
from jax.experimental import pallas as pl
from jax.experimental.pallas import tpu as pltpu
from jax.experimental.pallas import tpu_sc as plsc

N_NODES = 10000
N_EDGES = 320000
D = 128
C = 40

NC = 2
NS = 16
CHUNK = 64

E_PAD = 327680
N_ROWS = E_PAD // CHUNK
SEG_RPW = N_ROWS // NS
DCH = 128
DEG_ROWS = E_PAD // DCH
DEG_RPW = DEG_ROWS // (NC * NS)
N_ACC = 10240
ZROWS = N_ACC // NS
NBUF = 4
IDXBLK = 40
N_IDXBLK = SEG_RPW // IDXBLK
SUP_PER_BLK = IDXBLK // NBUF

_mesh = plsc.VectorSubcoreMesh(core_axis_name="c", subcore_axis_name="s")


def _zero_vmem_2d(ref, nrows):
    zv = jnp.zeros((16,), jnp.float32)

    def body(i, _):
        r = i // 8
        col = (i % 8) * 16
        ref[r, pl.ds(col, 16)] = zv
        return 0

    lax.fori_loop(0, nrows * 8, body, 0)


@functools.partial(
    pl.kernel,
    out_type=jax.ShapeDtypeStruct((2, N_ACC), jnp.float32),
    mesh=_mesh,
    scratch_types=[
        pltpu.VMEM_SHARED((N_ACC,), jnp.float32),
        pltpu.VMEM((640,), jnp.float32),
        pltpu.VMEM((DCH,), jnp.float32),
        pltpu.VMEM((DEG_RPW, DCH), jnp.int32),
        pltpu.SemaphoreType.DMA,
    ],
)
def _deg_kernel(dst2_hbm, out_hbm, acc, zbuf, ones_v, idx_sub, sem):
    c = lax.axis_index("c")
    s = lax.axis_index("s")

    def fill(i, _):
        zbuf[pl.ds(i * 16, 16)] = jnp.zeros((16,), jnp.float32)
        return 0

    lax.fori_loop(0, 40, fill, 0)

    def fill1(i, _):
        ones_v[pl.ds(i * 16, 16)] = jnp.ones((16,), jnp.float32)
        return 0

    lax.fori_loop(0, DCH // 16, fill1, 0)
    base_row = (c * NS + s) * DEG_RPW
    pltpu.sync_copy(dst2_hbm.at[pl.ds(base_row, DEG_RPW)], idx_sub)
    pltpu.sync_copy(zbuf, acc.at[pl.ds(s * ZROWS, ZROWS)])
    plsc.subcore_barrier()

    def fire(g, _):
        pltpu.async_copy(ones_v, acc.at[idx_sub.at[g]], sem, add=True)
        return 0

    lax.fori_loop(0, DEG_RPW, fire, 0)

    def drain(g, _):
        pltpu.make_async_copy(ones_v, acc.at[idx_sub.at[g]], sem).wait()
        return 0

    lax.fori_loop(0, DEG_RPW, drain, 0)
    plsc.subcore_barrier()

    pltpu.sync_copy(acc.at[pl.ds(s * ZROWS, ZROWS)], zbuf)
    pltpu.sync_copy(zbuf, out_hbm.at[c, pl.ds(s * ZROWS, ZROWS)])


@functools.partial(
    pl.kernel,
    out_type=(
        jax.ShapeDtypeStruct((N_ACC, D), jnp.float32),
        jax.ShapeDtypeStruct((N_ACC, D), jnp.float32),
    ),
    mesh=_mesh,
    scratch_types=[
        pltpu.VMEM_SHARED((N_ACC, D), jnp.float32),
        pltpu.VMEM((IDXBLK, CHUNK), jnp.int32),
        pltpu.VMEM((IDXBLK, CHUNK), jnp.int32),
        pltpu.VMEM((NBUF, CHUNK, D), jnp.float32),
        pltpu.SemaphoreType.DMA((NBUF,)),
        pltpu.SemaphoreType.DMA((NBUF,)),
    ],
)
def _segsum_kernel(h_hbm, hs_hbm, src2_hbm, dst2_hbm, s_out, t_out,
                   acc, src_sub, dst_sub, rows_v, gsem, ssem):
    c = lax.axis_index("c")
    s = lax.axis_index("s")

    _zero_vmem_2d(rows_v.at[0], CHUNK)
    for k in range(ZROWS // CHUNK):
        pltpu.sync_copy(rows_v.at[0],
                        acc.at[pl.ds(s * ZROWS + k * CHUNK, CHUNK)])
    plsc.subcore_barrier()

    def fire_gather(b, row):
        @pl.when(c == 0)
        def _():
            pltpu.async_copy(h_hbm.at[src_sub.at[row]], rows_v.at[b],
                             gsem.at[b])

        @pl.when(c == 1)
        def _():
            pltpu.async_copy(hs_hbm.at[src_sub.at[row]], rows_v.at[b],
                             gsem.at[b])

    def wait_gather(b, row):
        pltpu.make_async_copy(h_hbm.at[src_sub.at[row]], rows_v.at[b],
                              gsem.at[b]).wait()

    def fire_scatter(b, row):
        pltpu.async_copy(rows_v.at[b], acc.at[dst_sub.at[row]], ssem.at[b],
                         add=True)

    def wait_scatter(b, row):
        pltpu.make_async_copy(rows_v.at[b], acc.at[dst_sub.at[row]],
                              ssem.at[b]).wait()

    def block_body(blk, _):
        base = s * SEG_RPW + blk * IDXBLK
        pltpu.sync_copy(src2_hbm.at[pl.ds(base, IDXBLK)], src_sub)
        pltpu.sync_copy(dst2_hbm.at[pl.ds(base, IDXBLK)], dst_sub)
        for b in range(NBUF):
            fire_gather(b, b)

        def super_body(g, _):
            for b in range(NBUF):
                wait_gather(b, g * NBUF + b)
                fire_scatter(b, g * NBUF + b)

            @pl.when(g < SUP_PER_BLK - 1)
            def _():
                for b in range(NBUF):
                    wait_scatter(b, g * NBUF + b)
                    fire_gather(b, (g + 1) * NBUF + b)

            return 0

        lax.fori_loop(0, SUP_PER_BLK, super_body, 0)
        for b in range(NBUF):
            wait_scatter(b, (SUP_PER_BLK - 1) * NBUF + b)
        return 0

    lax.fori_loop(0, N_IDXBLK, block_body, 0)
    plsc.subcore_barrier()

    sl = pl.ds(s * ZROWS, ZROWS)

    @pl.when(c == 0)
    def _():
        pltpu.sync_copy(acc.at[sl], s_out.at[sl])

    @pl.when(c == 1)
    def _():
        pltpu.sync_copy(acc.at[sl], t_out.at[sl])


BN = 1000
_GRID = N_NODES // BN
_HP = jax.lax.Precision.HIGHEST


def _prep_body(x_ref, w_ref, b_ref, d0_ref, d1_ref, h_ref, hs_ref, rsd_ref,
               inv_ref):
    xb = x_ref[...]
    h = jnp.dot(xb, w_ref[...], preferred_element_type=jnp.float32,
                precision=_HP) + b_ref[...]
    deg = jnp.maximum(d0_ref[...] + d1_ref[...], 1.0)
    rsd = lax.rsqrt(deg)
    h_ref[...] = h
    hs_ref[...] = h * rsd
    rsd_ref[...] = rsd
    inv_ref[...] = 1.0 / deg


def _prep_call(x, lin1_w, lin1_b, d0, d1):
    return pl.pallas_call(
        _prep_body,
        grid=(_GRID,),
        in_specs=[
            pl.BlockSpec((BN, D), lambda i: (i, 0)),
            pl.BlockSpec((D, D), lambda i: (0, 0)),
            pl.BlockSpec((1, D), lambda i: (0, 0)),
            pl.BlockSpec((BN, 1), lambda i: (i, 0)),
            pl.BlockSpec((BN, 1), lambda i: (i, 0)),
        ],
        out_specs=[
            pl.BlockSpec((BN, D), lambda i: (i, 0)),
            pl.BlockSpec((BN, D), lambda i: (i, 0)),
            pl.BlockSpec((BN, 1), lambda i: (i, 0)),
            pl.BlockSpec((BN, 1), lambda i: (i, 0)),
        ],
        out_shape=[
            jax.ShapeDtypeStruct((N_NODES, D), jnp.float32),
            jax.ShapeDtypeStruct((N_NODES, D), jnp.float32),
            jax.ShapeDtypeStruct((N_NODES, 1), jnp.float32),
            jax.ShapeDtypeStruct((N_NODES, 1), jnp.float32),
        ],
    )(x, lin1_w, lin1_b.reshape(1, D), d0, d1)


def _elu(v):
    return jnp.where(v > 0, v, jnp.exp(jnp.minimum(v, 0.0)) - 1.0)


def _layer_body(h_ref, s_ref, t_ref, rsd_ref, inv_ref, gw_ref, gb_ref,
                sws_ref, swn_ref, sb_ref, ginw_ref, ginb_ref, naw_ref,
                hn_ref, hns_ref):
    h = h_ref[...]
    S = s_ref[...]
    T = t_ref[...]
    rsd = rsd_ref[...]
    inv = inv_ref[...]
    gcn = jnp.dot(T * rsd, gw_ref[...], preferred_element_type=jnp.float32,
                  precision=_HP) + gb_ref[...]
    sage = (jnp.dot(h, sws_ref[...], preferred_element_type=jnp.float32,
                    precision=_HP)
            + jnp.dot(S * inv, swn_ref[...], preferred_element_type=jnp.float32,
                      precision=_HP) + sb_ref[...])
    gin = jnp.dot(h + S, ginw_ref[...], preferred_element_type=jnp.float32,
                  precision=_HP) + ginb_ref[...]
    w0 = naw_ref[0]
    w1 = naw_ref[1]
    w2 = naw_ref[2]
    o = w0 * _elu(gcn) + w1 * _elu(sage) + w2 * _elu(gin)
    hn_ref[...] = o
    hns_ref[...] = o * rsd


def _layer_call(h, S, T, rsd, inv, gw, gb, sws, swn, sb, ginw, ginb, naw):
    full = lambda i: (0, 0)
    blk = lambda i: (i, 0)
    return pl.pallas_call(
        _layer_body,
        grid=(_GRID,),
        in_specs=[
            pl.BlockSpec((BN, D), blk),
            pl.BlockSpec((BN, D), blk),
            pl.BlockSpec((BN, D), blk),
            pl.BlockSpec((BN, 1), blk),
            pl.BlockSpec((BN, 1), blk),
            pl.BlockSpec((D, D), full),
            pl.BlockSpec((1, D), full),
            pl.BlockSpec((D, D), full),
            pl.BlockSpec((D, D), full),
            pl.BlockSpec((1, D), full),
            pl.BlockSpec((D, D), full),
            pl.BlockSpec((1, D), full),
            pl.BlockSpec(memory_space=pltpu.SMEM),
        ],
        out_specs=[
            pl.BlockSpec((BN, D), blk),
            pl.BlockSpec((BN, D), blk),
        ],
        out_shape=[
            jax.ShapeDtypeStruct((N_NODES, D), jnp.float32),
            jax.ShapeDtypeStruct((N_NODES, D), jnp.float32),
        ],
    )(h, S, T, rsd, inv, gw, gb.reshape(1, D), sws, swn, sb.reshape(1, D),
      ginw, ginb.reshape(1, D), naw)


def _final_body(x1_ref, x2_ref, x3_ref, lcw_ref, lcb_ref, ncw_ref, ncb_ref,
                sc_ref, law_ref, out_ref):
    c1 = sc_ref[0]
    c2 = sc_ref[1]
    a = x3_ref[...]
    b = c1 * x1_ref[...]
    c = c2 * x2_ref[...]
    l_max = jnp.maximum(jnp.maximum(a, b), c)
    l_mean = (a + b + c) * (1.0 / 3.0)
    lcw = lcw_ref[...]
    l_cat = (jnp.dot(a, lcw[0:D], preferred_element_type=jnp.float32,
                     precision=_HP)
             + jnp.dot(b, lcw[D:2 * D], preferred_element_type=jnp.float32,
                       precision=_HP)
             + jnp.dot(c, lcw[2 * D:3 * D], preferred_element_type=jnp.float32,
                       precision=_HP) + lcb_ref[...])
    la0 = law_ref[0]
    la1 = law_ref[1]
    la2 = law_ref[2]
    relu = lambda v: jnp.maximum(v, 0.0)
    x5 = la0 * relu(l_max) + la1 * relu(l_cat) + la2 * relu(l_mean)
    out_ref[...] = jnp.dot(x5, ncw_ref[...], preferred_element_type=jnp.float32,
                           precision=_HP) + ncb_ref[...]


def _final_call(x1, x2, x3, lc_w, lc_b, nc_w, nc_b, sc_c, la_w):
    full = lambda i: (0, 0)
    blk = lambda i: (i, 0)
    return pl.pallas_call(
        _final_body,
        grid=(_GRID,),
        in_specs=[
            pl.BlockSpec((BN, D), blk),
            pl.BlockSpec((BN, D), blk),
            pl.BlockSpec((BN, D), blk),
            pl.BlockSpec((3 * D, D), full),
            pl.BlockSpec((1, D), full),
            pl.BlockSpec((D, C), full),
            pl.BlockSpec((1, C), full),
            pl.BlockSpec(memory_space=pltpu.SMEM),
            pl.BlockSpec(memory_space=pltpu.SMEM),
        ],
        out_specs=pl.BlockSpec((BN, C), blk),
        out_shape=jax.ShapeDtypeStruct((N_NODES, C), jnp.float32),
    )(x1, x2, x3, lc_w, lc_b.reshape(1, D), nc_w, nc_b.reshape(1, C), sc_c,
      la_w)


def kernel(x, edge_index, lin1_w, lin1_b, gcn_w, gcn_b, sage_ws, sage_wn,
           sage_b, gin_w, gin_b, lc_w, lc_b, nc_w, nc_b, na_alphas, sc_alphas,
           la_alphas):
    na_w = jax.nn.softmax(na_alphas, axis=-1)
    sc_w = jax.nn.softmax(sc_alphas, axis=-1)
    la_w = jax.nn.softmax(la_alphas, axis=-1)

    src = edge_index[0]
    dst = edge_index[1]
    pad = E_PAD - N_EDGES
    iota = lax.iota(jnp.int32, pad)
    src_p = jnp.concatenate([src, iota % N_NODES]).reshape(N_ROWS, CHUNK)
    dst_flat = jnp.concatenate([dst, N_NODES + (iota % 8)])
    dst_p = dst_flat.reshape(N_ROWS, CHUNK)
    dst_p128 = dst_flat.reshape(DEG_ROWS, DCH)

    deg_pair = _deg_kernel(dst_p128)
    d0 = deg_pair[0, :N_NODES].reshape(N_NODES, 1)
    d1 = deg_pair[1, :N_NODES].reshape(N_NODES, 1)
    h, hs, rsd, inv = _prep_call(x, lin1_w, lin1_b, d0, d1)

    S, T = _segsum_kernel(h, hs, src_p, dst_p)
    x1, x1s = _layer_call(h, S[:N_NODES], T[:N_NODES], rsd, inv, gcn_w[0],
                          gcn_b[0], sage_ws[0], sage_wn[0], sage_b[0],
                          gin_w[0], gin_b[0], na_w[0])
    S, T = _segsum_kernel(x1, x1s, src_p, dst_p)
    x2, x2s = _layer_call(x1, S[:N_NODES], T[:N_NODES], rsd, inv, gcn_w[1],
                          gcn_b[1], sage_ws[1], sage_wn[1], sage_b[1],
                          gin_w[1], gin_b[1], na_w[1])
    S, T = _segsum_kernel(x2, x2s, src_p, dst_p)
    x3, _ = _layer_call(x2, S[:N_NODES], T[:N_NODES], rsd, inv, gcn_w[2],
                        gcn_b[2], sage_ws[2], sage_wn[2], sage_b[2],
                        gin_w[2], gin_b[2], na_w[2])

    sc_c = jnp.stack([sc_w[0, 1], sc_w[1, 1]])
    return _final_call(x1, x2, x3, lc_w, lc_b, nc_w, nc_b, sc_c, la_w[0])


# speedup vs baseline: 1.2129x; 1.0480x over previous
import functools

import jax
import jax.numpy as jnp
from jax import lax
from jax.experimental import pallas as pl
from jax.experimental.pallas import tpu as pltpu
from jax.experimental.pallas import tpu_sc as plsc

N_NODES = 10000
N_EDGES = 320000
D = 128
C = 40

NC = 2
NS = 16
CHUNK = 64

E_PAD = 327680
N_ROWS = E_PAD // CHUNK
SEG_RPW = N_ROWS // NS
DCH = 128
DEG_ROWS = E_PAD // DCH
DEG_RPW = DEG_ROWS // (NC * NS)
N_ACC = 10240
ZROWS = N_ACC // NS
NBUF = 4
IDXBLK = 40
N_IDXBLK = SEG_RPW // IDXBLK
SUP_PER_BLK = IDXBLK // NBUF

_mesh = plsc.VectorSubcoreMesh(core_axis_name="c", subcore_axis_name="s")


def _zero_vmem_2d(ref, nrows):
    zv = jnp.zeros((16,), jnp.float32)

    def body(i, _):
        r = i // 8
        col = (i % 8) * 16
        ref[r, pl.ds(col, 16)] = zv
        return 0

    lax.fori_loop(0, nrows * 8, body, 0)


@functools.partial(
    pl.kernel,
    out_type=jax.ShapeDtypeStruct((2, N_ACC), jnp.float32),
    mesh=_mesh,
    scratch_types=[
        pltpu.VMEM_SHARED((N_ACC,), jnp.float32),
        pltpu.VMEM((640,), jnp.float32),
        pltpu.VMEM((DCH,), jnp.float32),
        pltpu.VMEM((DEG_RPW, DCH), jnp.int32),
        pltpu.SemaphoreType.DMA,
    ],
)
def _deg_kernel(dst2_hbm, out_hbm, acc, zbuf, ones_v, idx_sub, sem):
    c = lax.axis_index("c")
    s = lax.axis_index("s")

    def fill(i, _):
        zbuf[pl.ds(i * 16, 16)] = jnp.zeros((16,), jnp.float32)
        return 0

    lax.fori_loop(0, 40, fill, 0)

    def fill1(i, _):
        ones_v[pl.ds(i * 16, 16)] = jnp.ones((16,), jnp.float32)
        return 0

    lax.fori_loop(0, DCH // 16, fill1, 0)
    base_row = (c * NS + s) * DEG_RPW
    pltpu.sync_copy(dst2_hbm.at[pl.ds(base_row, DEG_RPW)], idx_sub)
    pltpu.sync_copy(zbuf, acc.at[pl.ds(s * ZROWS, ZROWS)])
    plsc.subcore_barrier()

    def fire(g, _):
        pltpu.async_copy(ones_v, acc.at[idx_sub.at[g]], sem, add=True)
        return 0

    lax.fori_loop(0, DEG_RPW, fire, 0)

    def drain(g, _):
        pltpu.make_async_copy(ones_v, acc.at[idx_sub.at[g]], sem).wait()
        return 0

    lax.fori_loop(0, DEG_RPW, drain, 0)
    plsc.subcore_barrier()

    pltpu.sync_copy(acc.at[pl.ds(s * ZROWS, ZROWS)], zbuf)
    pltpu.sync_copy(zbuf, out_hbm.at[c, pl.ds(s * ZROWS, ZROWS)])


@functools.partial(
    pl.kernel,
    out_type=(
        jax.ShapeDtypeStruct((N_ACC, D), jnp.float32),
        jax.ShapeDtypeStruct((N_ACC, D), jnp.float32),
    ),
    mesh=_mesh,
    scratch_types=[
        pltpu.VMEM_SHARED((N_ACC, D), jnp.float32),
        pltpu.VMEM((IDXBLK, CHUNK), jnp.int32),
        pltpu.VMEM((IDXBLK, CHUNK), jnp.int32),
        pltpu.VMEM((NBUF, CHUNK, D), jnp.float32),
        pltpu.SemaphoreType.DMA((NBUF,)),
        pltpu.SemaphoreType.DMA((NBUF,)),
    ],
)
def _segsum_kernel(h_hbm, hs_hbm, src2_hbm, dst2_hbm, s_out, t_out,
                   acc, src_sub, dst_sub, rows_v, gsem, ssem):
    c = lax.axis_index("c")
    s = lax.axis_index("s")

    _zero_vmem_2d(rows_v.at[0], CHUNK)
    for k in range(ZROWS // CHUNK):
        pltpu.sync_copy(rows_v.at[0],
                        acc.at[pl.ds(s * ZROWS + k * CHUNK, CHUNK)])
    plsc.subcore_barrier()

    def fire_gather(b, row):
        @pl.when(c == 0)
        def _():
            pltpu.async_copy(h_hbm.at[src_sub.at[row]], rows_v.at[b],
                             gsem.at[b])

        @pl.when(c == 1)
        def _():
            pltpu.async_copy(hs_hbm.at[src_sub.at[row]], rows_v.at[b],
                             gsem.at[b])

    def wait_gather(b, row):
        pltpu.make_async_copy(h_hbm.at[src_sub.at[row]], rows_v.at[b],
                              gsem.at[b]).wait()

    def fire_scatter(b, row):
        pltpu.async_copy(rows_v.at[b], acc.at[dst_sub.at[row]], ssem.at[b],
                         add=True)

    def wait_scatter(b, row):
        pltpu.make_async_copy(rows_v.at[b], acc.at[dst_sub.at[row]],
                              ssem.at[b]).wait()

    def block_body(blk, _):
        base = s * SEG_RPW + blk * IDXBLK
        pltpu.sync_copy(src2_hbm.at[pl.ds(base, IDXBLK)], src_sub)
        pltpu.sync_copy(dst2_hbm.at[pl.ds(base, IDXBLK)], dst_sub)
        for b in range(NBUF):
            fire_gather(b, b)

        def super_body(g, _):
            for b in range(NBUF):
                wait_gather(b, g * NBUF + b)
                fire_scatter(b, g * NBUF + b)

            @pl.when(g < SUP_PER_BLK - 1)
            def _():
                for b in range(NBUF):
                    wait_scatter(b, g * NBUF + b)
                    fire_gather(b, (g + 1) * NBUF + b)

            return 0

        lax.fori_loop(0, SUP_PER_BLK, super_body, 0)
        for b in range(NBUF):
            wait_scatter(b, (SUP_PER_BLK - 1) * NBUF + b)
        return 0

    lax.fori_loop(0, N_IDXBLK, block_body, 0)
    plsc.subcore_barrier()

    sl = pl.ds(s * ZROWS, ZROWS)

    @pl.when(c == 0)
    def _():
        pltpu.sync_copy(acc.at[sl], s_out.at[sl])

    @pl.when(c == 1)
    def _():
        pltpu.sync_copy(acc.at[sl], t_out.at[sl])


BN = 1000
_GRID = N_NODES // BN
_HP = jax.lax.Precision.HIGHEST


def _prep_body(x_ref, w_ref, b_ref, d0_ref, d1_ref, h_ref, hs_ref, rsd_ref,
               inv_ref):
    xb = x_ref[...]
    h = jnp.dot(xb, w_ref[...], preferred_element_type=jnp.float32,
                precision=_HP) + b_ref[...]
    deg = jnp.maximum(d0_ref[...] + d1_ref[...], 1.0)
    rsd = lax.rsqrt(deg)
    h_ref[...] = h
    hs_ref[...] = h * rsd
    rsd_ref[...] = rsd
    inv_ref[...] = 1.0 / deg


def _prep_call(x, lin1_w, lin1_b, d0, d1):
    return pl.pallas_call(
        _prep_body,
        grid=(_GRID,),
        in_specs=[
            pl.BlockSpec((BN, D), lambda i: (i, 0)),
            pl.BlockSpec((D, D), lambda i: (0, 0)),
            pl.BlockSpec((1, D), lambda i: (0, 0)),
            pl.BlockSpec((BN, 1), lambda i: (i, 0)),
            pl.BlockSpec((BN, 1), lambda i: (i, 0)),
        ],
        out_specs=[
            pl.BlockSpec((BN, D), lambda i: (i, 0)),
            pl.BlockSpec((BN, D), lambda i: (i, 0)),
            pl.BlockSpec((BN, 1), lambda i: (i, 0)),
            pl.BlockSpec((BN, 1), lambda i: (i, 0)),
        ],
        out_shape=[
            jax.ShapeDtypeStruct((N_NODES, D), jnp.float32),
            jax.ShapeDtypeStruct((N_NODES, D), jnp.float32),
            jax.ShapeDtypeStruct((N_NODES, 1), jnp.float32),
            jax.ShapeDtypeStruct((N_NODES, 1), jnp.float32),
        ],
    )(x, lin1_w, lin1_b.reshape(1, D), d0, d1)


def _elu(v):
    return jnp.where(v > 0, v, jnp.exp(jnp.minimum(v, 0.0)) - 1.0)


def _layer_pre_body(h_ref, sws_ref, sb_ref, ginw_ref, pre_ref):
    h = h_ref[...]
    pre_ref[0] = jnp.dot(h, sws_ref[...], preferred_element_type=jnp.float32,
                         precision=_HP) + sb_ref[...]
    pre_ref[1] = jnp.dot(h, ginw_ref[...], preferred_element_type=jnp.float32,
                         precision=_HP)


def _layer_pre_call(h, sws, sb, ginw):
    full = lambda i: (0, 0)
    blk = lambda i: (i, 0)
    return pl.pallas_call(
        _layer_pre_body,
        grid=(_GRID,),
        in_specs=[
            pl.BlockSpec((BN, D), blk),
            pl.BlockSpec((D, D), full),
            pl.BlockSpec((1, D), full),
            pl.BlockSpec((D, D), full),
        ],
        out_specs=pl.BlockSpec((2, BN, D), lambda i: (0, i, 0)),
        out_shape=jax.ShapeDtypeStruct((2, N_NODES, D), jnp.float32),
    )(h, sws, sb.reshape(1, D), ginw)


def _layer_body(h_ref, s_ref, t_ref, rsd_ref, inv_ref, pre_ref, gw_ref,
                gb_ref, swn_ref, ginw_ref, ginb_ref, naw_ref,
                hn_ref, hns_ref):
    h = h_ref[...]
    S = s_ref[...]
    T = t_ref[...]
    rsd = rsd_ref[...]
    inv = inv_ref[...]
    gcn = jnp.dot(T * rsd, gw_ref[...], preferred_element_type=jnp.float32,
                  precision=_HP) + gb_ref[...]
    sage = (pre_ref[0]
            + jnp.dot(S * inv, swn_ref[...], preferred_element_type=jnp.float32,
                      precision=_HP))
    gin = (pre_ref[1]
           + jnp.dot(S, ginw_ref[...], preferred_element_type=jnp.float32,
                     precision=_HP) + ginb_ref[...])
    w0 = naw_ref[0]
    w1 = naw_ref[1]
    w2 = naw_ref[2]
    o = w0 * _elu(gcn) + w1 * _elu(sage) + w2 * _elu(gin)
    hn_ref[...] = o
    hns_ref[...] = o * rsd


def _layer_call(h, S, T, rsd, inv, pre, gw, gb, swn, ginw, ginb, naw):
    full = lambda i: (0, 0)
    blk = lambda i: (i, 0)
    return pl.pallas_call(
        _layer_body,
        grid=(_GRID,),
        in_specs=[
            pl.BlockSpec((BN, D), blk),
            pl.BlockSpec((BN, D), blk),
            pl.BlockSpec((BN, D), blk),
            pl.BlockSpec((BN, 1), blk),
            pl.BlockSpec((BN, 1), blk),
            pl.BlockSpec((2, BN, D), lambda i: (0, i, 0)),
            pl.BlockSpec((D, D), full),
            pl.BlockSpec((1, D), full),
            pl.BlockSpec((D, D), full),
            pl.BlockSpec((D, D), full),
            pl.BlockSpec((1, D), full),
            pl.BlockSpec(memory_space=pltpu.SMEM),
        ],
        out_specs=[
            pl.BlockSpec((BN, D), blk),
            pl.BlockSpec((BN, D), blk),
        ],
        out_shape=[
            jax.ShapeDtypeStruct((N_NODES, D), jnp.float32),
            jax.ShapeDtypeStruct((N_NODES, D), jnp.float32),
        ],
    )(h, S, T, rsd, inv, pre, gw, gb.reshape(1, D), swn,
      ginw, ginb.reshape(1, D), naw)


def _final_body(x1_ref, x2_ref, x3_ref, lcw_ref, lcb_ref, ncw_ref, ncb_ref,
                sc_ref, law_ref, out_ref):
    c1 = sc_ref[0]
    c2 = sc_ref[1]
    a = x3_ref[...]
    b = c1 * x1_ref[...]
    c = c2 * x2_ref[...]
    l_max = jnp.maximum(jnp.maximum(a, b), c)
    l_mean = (a + b + c) * (1.0 / 3.0)
    lcw = lcw_ref[...]
    l_cat = (jnp.dot(a, lcw[0:D], preferred_element_type=jnp.float32,
                     precision=_HP)
             + jnp.dot(b, lcw[D:2 * D], preferred_element_type=jnp.float32,
                       precision=_HP)
             + jnp.dot(c, lcw[2 * D:3 * D], preferred_element_type=jnp.float32,
                       precision=_HP) + lcb_ref[...])
    la0 = law_ref[0]
    la1 = law_ref[1]
    la2 = law_ref[2]
    relu = lambda v: jnp.maximum(v, 0.0)
    x5 = la0 * relu(l_max) + la1 * relu(l_cat) + la2 * relu(l_mean)
    out_ref[...] = jnp.dot(x5, ncw_ref[...], preferred_element_type=jnp.float32,
                           precision=_HP) + ncb_ref[...]


def _final_call(x1, x2, x3, lc_w, lc_b, nc_w, nc_b, sc_c, la_w):
    full = lambda i: (0, 0)
    blk = lambda i: (i, 0)
    return pl.pallas_call(
        _final_body,
        grid=(_GRID,),
        in_specs=[
            pl.BlockSpec((BN, D), blk),
            pl.BlockSpec((BN, D), blk),
            pl.BlockSpec((BN, D), blk),
            pl.BlockSpec((3 * D, D), full),
            pl.BlockSpec((1, D), full),
            pl.BlockSpec((D, C), full),
            pl.BlockSpec((1, C), full),
            pl.BlockSpec(memory_space=pltpu.SMEM),
            pl.BlockSpec(memory_space=pltpu.SMEM),
        ],
        out_specs=pl.BlockSpec((BN, C), blk),
        out_shape=jax.ShapeDtypeStruct((N_NODES, C), jnp.float32),
    )(x1, x2, x3, lc_w, lc_b.reshape(1, D), nc_w, nc_b.reshape(1, C), sc_c,
      la_w)


def kernel(x, edge_index, lin1_w, lin1_b, gcn_w, gcn_b, sage_ws, sage_wn,
           sage_b, gin_w, gin_b, lc_w, lc_b, nc_w, nc_b, na_alphas, sc_alphas,
           la_alphas):
    na_w = jax.nn.softmax(na_alphas, axis=-1)
    sc_w = jax.nn.softmax(sc_alphas, axis=-1)
    la_w = jax.nn.softmax(la_alphas, axis=-1)

    src = edge_index[0]
    dst = edge_index[1]
    pad = E_PAD - N_EDGES
    iota = lax.iota(jnp.int32, pad)
    src_p = jnp.concatenate([src, iota % N_NODES]).reshape(N_ROWS, CHUNK)
    dst_flat = jnp.concatenate([dst, N_NODES + (iota % 8)])
    dst_p = dst_flat.reshape(N_ROWS, CHUNK)
    dst_p128 = dst_flat.reshape(DEG_ROWS, DCH)

    deg_pair = _deg_kernel(dst_p128)
    d0 = deg_pair[0, :N_NODES].reshape(N_NODES, 1)
    d1 = deg_pair[1, :N_NODES].reshape(N_NODES, 1)
    h, hs, rsd, inv = _prep_call(x, lin1_w, lin1_b, d0, d1)

    S, T = _segsum_kernel(h, hs, src_p, dst_p)
    pre1 = _layer_pre_call(h, sage_ws[0], sage_b[0], gin_w[0])
    x1, x1s = _layer_call(h, S, T, rsd, inv, pre1, gcn_w[0], gcn_b[0],
                          sage_wn[0], gin_w[0], gin_b[0], na_w[0])
    S, T = _segsum_kernel(x1, x1s, src_p, dst_p)
    pre2 = _layer_pre_call(x1, sage_ws[1], sage_b[1], gin_w[1])
    x2, x2s = _layer_call(x1, S, T, rsd, inv, pre2, gcn_w[1], gcn_b[1],
                          sage_wn[1], gin_w[1], gin_b[1], na_w[1])
    S, T = _segsum_kernel(x2, x2s, src_p, dst_p)
    pre3 = _layer_pre_call(x2, sage_ws[2], sage_b[2], gin_w[2])
    x3, _ = _layer_call(x2, S, T, rsd, inv, pre3, gcn_w[2], gcn_b[2],
                        sage_wn[2], gin_w[2], gin_b[2], na_w[2])

    sc_c = jnp.stack([sc_w[0, 1], sc_w[1, 1]])
    return _final_call(x1, x2, x3, lc_w, lc_b, nc_w, nc_b, sc_c, la_w[0])


# speedup vs baseline: 1.2242x; 1.0093x over previous
import functools

import jax
import jax.numpy as jnp
from jax import lax
from jax.experimental import pallas as pl
from jax.experimental.pallas import tpu as pltpu
from jax.experimental.pallas import tpu_sc as plsc

N_NODES = 10000
N_EDGES = 320000
D = 128
C = 40

NC = 2
NS = 16
CHUNK = 64

E_PAD = 327680
N_ROWS = E_PAD // CHUNK
SEG_RPW = N_ROWS // NS
DCH = 128
DEG_ROWS = E_PAD // DCH
DEG_RPW = DEG_ROWS // (NC * NS)
N_ACC = 10240
ZROWS = N_ACC // NS
NBUF = 4
IDXBLK = 40
N_IDXBLK = SEG_RPW // IDXBLK
SUP_PER_BLK = IDXBLK // NBUF

_mesh = plsc.VectorSubcoreMesh(core_axis_name="c", subcore_axis_name="s")


def _zero_vmem_2d(ref, nrows):
    zv = jnp.zeros((16,), jnp.float32)

    def body(i, _):
        r = i // 8
        col = (i % 8) * 16
        ref[r, pl.ds(col, 16)] = zv
        return 0

    lax.fori_loop(0, nrows * 8, body, 0)


@functools.partial(
    pl.kernel,
    out_type=jax.ShapeDtypeStruct((2, N_ACC), jnp.float32),
    mesh=_mesh,
    scratch_types=[
        pltpu.VMEM_SHARED((N_ACC,), jnp.float32),
        pltpu.VMEM((640,), jnp.float32),
        pltpu.VMEM((DCH,), jnp.float32),
        pltpu.VMEM((DEG_RPW, DCH), jnp.int32),
        pltpu.SemaphoreType.DMA,
    ],
)
def _deg_kernel(dst2_hbm, out_hbm, acc, zbuf, ones_v, idx_sub, sem):
    c = lax.axis_index("c")
    s = lax.axis_index("s")

    def fill(i, _):
        zbuf[pl.ds(i * 16, 16)] = jnp.zeros((16,), jnp.float32)
        return 0

    lax.fori_loop(0, 40, fill, 0)

    def fill1(i, _):
        ones_v[pl.ds(i * 16, 16)] = jnp.ones((16,), jnp.float32)
        return 0

    lax.fori_loop(0, DCH // 16, fill1, 0)
    base_row = (c * NS + s) * DEG_RPW
    pltpu.sync_copy(dst2_hbm.at[pl.ds(base_row, DEG_RPW)], idx_sub)
    pltpu.sync_copy(zbuf, acc.at[pl.ds(s * ZROWS, ZROWS)])
    plsc.subcore_barrier()

    def fire(g, _):
        pltpu.async_copy(ones_v, acc.at[idx_sub.at[g]], sem, add=True)
        return 0

    lax.fori_loop(0, DEG_RPW, fire, 0)

    def drain(g, _):
        pltpu.make_async_copy(ones_v, acc.at[idx_sub.at[g]], sem).wait()
        return 0

    lax.fori_loop(0, DEG_RPW, drain, 0)
    plsc.subcore_barrier()

    pltpu.sync_copy(acc.at[pl.ds(s * ZROWS, ZROWS)], zbuf)
    pltpu.sync_copy(zbuf, out_hbm.at[c, pl.ds(s * ZROWS, ZROWS)])


@functools.partial(
    pl.kernel,
    out_type=(
        jax.ShapeDtypeStruct((N_ACC, D), jnp.float32),
        jax.ShapeDtypeStruct((N_ACC, D), jnp.float32),
    ),
    mesh=_mesh,
    scratch_types=[
        pltpu.VMEM_SHARED((N_ACC, D), jnp.float32),
        pltpu.VMEM((IDXBLK, CHUNK), jnp.int32),
        pltpu.VMEM((IDXBLK, CHUNK), jnp.int32),
        pltpu.VMEM((NBUF, CHUNK, D), jnp.float32),
        pltpu.SemaphoreType.DMA((NBUF,)),
        pltpu.SemaphoreType.DMA((NBUF,)),
    ],
)
def _segsum_kernel(h_hbm, hs_hbm, src2_hbm, dst2_hbm, s_out, t_out,
                   acc, src_sub, dst_sub, rows_v, gsem, ssem):
    c = lax.axis_index("c")
    s = lax.axis_index("s")

    _zero_vmem_2d(rows_v.at[0], CHUNK)
    for k in range(ZROWS // CHUNK):
        pltpu.sync_copy(rows_v.at[0],
                        acc.at[pl.ds(s * ZROWS + k * CHUNK, CHUNK)])
    plsc.subcore_barrier()

    def fire_gather(b, row):
        @pl.when(c == 0)
        def _():
            pltpu.async_copy(h_hbm.at[src_sub.at[row]], rows_v.at[b],
                             gsem.at[b])

        @pl.when(c == 1)
        def _():
            pltpu.async_copy(hs_hbm.at[src_sub.at[row]], rows_v.at[b],
                             gsem.at[b])

    def wait_gather(b, row):
        pltpu.make_async_copy(h_hbm.at[src_sub.at[row]], rows_v.at[b],
                              gsem.at[b]).wait()

    def fire_scatter(b, row):
        pltpu.async_copy(rows_v.at[b], acc.at[dst_sub.at[row]], ssem.at[b],
                         add=True)

    def wait_scatter(b, row):
        pltpu.make_async_copy(rows_v.at[b], acc.at[dst_sub.at[row]],
                              ssem.at[b]).wait()

    def block_body(blk, _):
        base = s * SEG_RPW + blk * IDXBLK
        pltpu.sync_copy(src2_hbm.at[pl.ds(base, IDXBLK)], src_sub)
        pltpu.sync_copy(dst2_hbm.at[pl.ds(base, IDXBLK)], dst_sub)
        for b in range(NBUF):
            fire_gather(b, b)

        def super_body(g, _):
            for b in range(NBUF):
                wait_gather(b, g * NBUF + b)
                fire_scatter(b, g * NBUF + b)

            @pl.when(g < SUP_PER_BLK - 1)
            def _():
                for b in range(NBUF):
                    wait_scatter(b, g * NBUF + b)
                    fire_gather(b, (g + 1) * NBUF + b)

            return 0

        lax.fori_loop(0, SUP_PER_BLK, super_body, 0)
        for b in range(NBUF):
            wait_scatter(b, (SUP_PER_BLK - 1) * NBUF + b)
        return 0

    lax.fori_loop(0, N_IDXBLK, block_body, 0)
    plsc.subcore_barrier()

    sl = pl.ds(s * ZROWS, ZROWS)

    @pl.when(c == 0)
    def _():
        pltpu.sync_copy(acc.at[sl], s_out.at[sl])

    @pl.when(c == 1)
    def _():
        pltpu.sync_copy(acc.at[sl], t_out.at[sl])


BN = 1000
_GRID = N_NODES // BN
_HP = jax.lax.Precision.HIGHEST


def _lin1_body(x_ref, w_ref, b_ref, h_ref):
    h_ref[...] = jnp.dot(x_ref[...], w_ref[...],
                         preferred_element_type=jnp.float32,
                         precision=_HP) + b_ref[...]


def _lin1_call(x, lin1_w, lin1_b):
    return pl.pallas_call(
        _lin1_body,
        grid=(_GRID,),
        in_specs=[
            pl.BlockSpec((BN, D), lambda i: (i, 0)),
            pl.BlockSpec((D, D), lambda i: (0, 0)),
            pl.BlockSpec((1, D), lambda i: (0, 0)),
        ],
        out_specs=pl.BlockSpec((BN, D), lambda i: (i, 0)),
        out_shape=jax.ShapeDtypeStruct((N_NODES, D), jnp.float32),
    )(x, lin1_w, lin1_b.reshape(1, D))


def _prep_body(h_ref, d0_ref, d1_ref, hs_ref, rsd_ref, inv_ref):
    deg = jnp.maximum(d0_ref[...] + d1_ref[...], 1.0)
    rsd = lax.rsqrt(deg)
    hs_ref[...] = h_ref[...] * rsd
    rsd_ref[...] = rsd
    inv_ref[...] = 1.0 / deg


def _prep_call(h, d0, d1):
    return pl.pallas_call(
        _prep_body,
        grid=(_GRID,),
        in_specs=[
            pl.BlockSpec((BN, D), lambda i: (i, 0)),
            pl.BlockSpec((BN, 1), lambda i: (i, 0)),
            pl.BlockSpec((BN, 1), lambda i: (i, 0)),
        ],
        out_specs=[
            pl.BlockSpec((BN, D), lambda i: (i, 0)),
            pl.BlockSpec((BN, 1), lambda i: (i, 0)),
            pl.BlockSpec((BN, 1), lambda i: (i, 0)),
        ],
        out_shape=[
            jax.ShapeDtypeStruct((N_NODES, D), jnp.float32),
            jax.ShapeDtypeStruct((N_NODES, 1), jnp.float32),
            jax.ShapeDtypeStruct((N_NODES, 1), jnp.float32),
        ],
    )(h, d0, d1)


def _elu(v):
    return jnp.where(v > 0, v, jnp.exp(jnp.minimum(v, 0.0)) - 1.0)


def _layer_pre_body(h_ref, sws_ref, sb_ref, ginw_ref, pre_ref):
    h = h_ref[...]
    pre_ref[0] = jnp.dot(h, sws_ref[...], preferred_element_type=jnp.float32,
                         precision=_HP) + sb_ref[...]
    pre_ref[1] = jnp.dot(h, ginw_ref[...], preferred_element_type=jnp.float32,
                         precision=_HP)


def _layer_pre_call(h, sws, sb, ginw):
    full = lambda i: (0, 0)
    blk = lambda i: (i, 0)
    return pl.pallas_call(
        _layer_pre_body,
        grid=(_GRID,),
        in_specs=[
            pl.BlockSpec((BN, D), blk),
            pl.BlockSpec((D, D), full),
            pl.BlockSpec((1, D), full),
            pl.BlockSpec((D, D), full),
        ],
        out_specs=pl.BlockSpec((2, BN, D), lambda i: (0, i, 0)),
        out_shape=jax.ShapeDtypeStruct((2, N_NODES, D), jnp.float32),
    )(h, sws, sb.reshape(1, D), ginw)


def _layer_body(h_ref, s_ref, t_ref, rsd_ref, inv_ref, pre_ref, gw_ref,
                gb_ref, swn_ref, ginw_ref, ginb_ref, naw_ref,
                hn_ref, hns_ref):
    h = h_ref[...]
    S = s_ref[...]
    T = t_ref[...]
    rsd = rsd_ref[...]
    inv = inv_ref[...]
    gcn = jnp.dot(T * rsd, gw_ref[...], preferred_element_type=jnp.float32,
                  precision=_HP) + gb_ref[...]
    sage = (pre_ref[0]
            + jnp.dot(S * inv, swn_ref[...], preferred_element_type=jnp.float32,
                      precision=_HP))
    gin = (pre_ref[1]
           + jnp.dot(S, ginw_ref[...], preferred_element_type=jnp.float32,
                     precision=_HP) + ginb_ref[...])
    w0 = naw_ref[0]
    w1 = naw_ref[1]
    w2 = naw_ref[2]
    o = w0 * _elu(gcn) + w1 * _elu(sage) + w2 * _elu(gin)
    hn_ref[...] = o
    hns_ref[...] = o * rsd


def _layer_call(h, S, T, rsd, inv, pre, gw, gb, swn, ginw, ginb, naw):
    full = lambda i: (0, 0)
    blk = lambda i: (i, 0)
    return pl.pallas_call(
        _layer_body,
        grid=(_GRID,),
        in_specs=[
            pl.BlockSpec((BN, D), blk),
            pl.BlockSpec((BN, D), blk),
            pl.BlockSpec((BN, D), blk),
            pl.BlockSpec((BN, 1), blk),
            pl.BlockSpec((BN, 1), blk),
            pl.BlockSpec((2, BN, D), lambda i: (0, i, 0)),
            pl.BlockSpec((D, D), full),
            pl.BlockSpec((1, D), full),
            pl.BlockSpec((D, D), full),
            pl.BlockSpec((D, D), full),
            pl.BlockSpec((1, D), full),
            pl.BlockSpec(memory_space=pltpu.SMEM),
        ],
        out_specs=[
            pl.BlockSpec((BN, D), blk),
            pl.BlockSpec((BN, D), blk),
        ],
        out_shape=[
            jax.ShapeDtypeStruct((N_NODES, D), jnp.float32),
            jax.ShapeDtypeStruct((N_NODES, D), jnp.float32),
        ],
    )(h, S, T, rsd, inv, pre, gw, gb.reshape(1, D), swn,
      ginw, ginb.reshape(1, D), naw)


def _final_pre_body(x1_ref, x2_ref, lcw_ref, lcb_ref, sc_ref, pre_ref):
    c1 = sc_ref[0]
    c2 = sc_ref[1]
    b = c1 * x1_ref[...]
    c = c2 * x2_ref[...]
    lcw = lcw_ref[...]
    pre_ref[0] = (jnp.dot(b, lcw[D:2 * D], preferred_element_type=jnp.float32,
                          precision=_HP)
                  + jnp.dot(c, lcw[2 * D:3 * D],
                            preferred_element_type=jnp.float32,
                            precision=_HP) + lcb_ref[...])
    pre_ref[1] = jnp.maximum(b, c)
    pre_ref[2] = b + c


def _final_pre_call(x1, x2, lc_w, lc_b, sc_c):
    full = lambda i: (0, 0)
    blk = lambda i: (i, 0)
    return pl.pallas_call(
        _final_pre_body,
        grid=(_GRID,),
        in_specs=[
            pl.BlockSpec((BN, D), blk),
            pl.BlockSpec((BN, D), blk),
            pl.BlockSpec((3 * D, D), full),
            pl.BlockSpec((1, D), full),
            pl.BlockSpec(memory_space=pltpu.SMEM),
        ],
        out_specs=pl.BlockSpec((3, BN, D), lambda i: (0, i, 0)),
        out_shape=jax.ShapeDtypeStruct((3, N_NODES, D), jnp.float32),
    )(x1, x2, lc_w, lc_b.reshape(1, D), sc_c)


def _final_body(x3_ref, pre_ref, lcw_ref, ncw_ref, ncb_ref, law_ref, out_ref):
    a = x3_ref[...]
    l_max = jnp.maximum(a, pre_ref[1])
    l_mean = (a + pre_ref[2]) * (1.0 / 3.0)
    l_cat = (jnp.dot(a, lcw_ref[...], preferred_element_type=jnp.float32,
                     precision=_HP) + pre_ref[0])
    la0 = law_ref[0]
    la1 = law_ref[1]
    la2 = law_ref[2]
    relu = lambda v: jnp.maximum(v, 0.0)
    x5 = la0 * relu(l_max) + la1 * relu(l_cat) + la2 * relu(l_mean)
    out_ref[...] = jnp.dot(x5, ncw_ref[...], preferred_element_type=jnp.float32,
                           precision=_HP) + ncb_ref[...]


def _final_call(x3, pre, lc_w, nc_w, nc_b, la_w):
    full = lambda i: (0, 0)
    blk = lambda i: (i, 0)
    return pl.pallas_call(
        _final_body,
        grid=(_GRID,),
        in_specs=[
            pl.BlockSpec((BN, D), blk),
            pl.BlockSpec((3, BN, D), lambda i: (0, i, 0)),
            pl.BlockSpec((D, D), full),
            pl.BlockSpec((D, C), full),
            pl.BlockSpec((1, C), full),
            pl.BlockSpec(memory_space=pltpu.SMEM),
        ],
        out_specs=pl.BlockSpec((BN, C), blk),
        out_shape=jax.ShapeDtypeStruct((N_NODES, C), jnp.float32),
    )(x3, pre, lc_w[0:D], nc_w, nc_b.reshape(1, C), la_w)


def kernel(x, edge_index, lin1_w, lin1_b, gcn_w, gcn_b, sage_ws, sage_wn,
           sage_b, gin_w, gin_b, lc_w, lc_b, nc_w, nc_b, na_alphas, sc_alphas,
           la_alphas):
    na_w = jax.nn.softmax(na_alphas, axis=-1)
    sc_w = jax.nn.softmax(sc_alphas, axis=-1)
    la_w = jax.nn.softmax(la_alphas, axis=-1)

    src = edge_index[0]
    dst = edge_index[1]
    pad = E_PAD - N_EDGES
    iota = lax.iota(jnp.int32, pad)
    src_p = jnp.concatenate([src, iota % N_NODES]).reshape(N_ROWS, CHUNK)
    dst_flat = jnp.concatenate([dst, N_NODES + (iota % 8)])
    dst_p = dst_flat.reshape(N_ROWS, CHUNK)
    dst_p128 = dst_flat.reshape(DEG_ROWS, DCH)

    deg_pair = _deg_kernel(dst_p128)
    h = _lin1_call(x, lin1_w, lin1_b)
    d0 = deg_pair[0, :N_NODES].reshape(N_NODES, 1)
    d1 = deg_pair[1, :N_NODES].reshape(N_NODES, 1)
    hs, rsd, inv = _prep_call(h, d0, d1)

    S, T = _segsum_kernel(h, hs, src_p, dst_p)
    pre1 = _layer_pre_call(h, sage_ws[0], sage_b[0], gin_w[0])
    x1, x1s = _layer_call(h, S, T, rsd, inv, pre1, gcn_w[0], gcn_b[0],
                          sage_wn[0], gin_w[0], gin_b[0], na_w[0])
    S, T = _segsum_kernel(x1, x1s, src_p, dst_p)
    pre2 = _layer_pre_call(x1, sage_ws[1], sage_b[1], gin_w[1])
    x2, x2s = _layer_call(x1, S, T, rsd, inv, pre2, gcn_w[1], gcn_b[1],
                          sage_wn[1], gin_w[1], gin_b[1], na_w[1])
    sc_c = jnp.stack([sc_w[0, 1], sc_w[1, 1]])
    S, T = _segsum_kernel(x2, x2s, src_p, dst_p)
    pre3 = _layer_pre_call(x2, sage_ws[2], sage_b[2], gin_w[2])
    fpre = _final_pre_call(x1, x2, lc_w, lc_b, sc_c)
    x3, _ = _layer_call(x2, S, T, rsd, inv, pre3, gcn_w[2], gcn_b[2],
                        sage_wn[2], gin_w[2], gin_b[2], na_w[2])

    return _final_call(x3, fpre, lc_w, nc_w, nc_b, la_w[0])


# speedup vs baseline: 1.2738x; 1.0405x over previous
import functools

import jax
import jax.numpy as jnp
from jax import lax
from jax.experimental import pallas as pl
from jax.experimental.pallas import tpu as pltpu
from jax.experimental.pallas import tpu_sc as plsc

N_NODES = 10000
N_EDGES = 320000
D = 128
C = 40

NC = 2
NS = 16
CHUNK = 64

E_PAD = 327680
N_ROWS = E_PAD // CHUNK
SEG_RPW = N_ROWS // NS
DCH = 128
DEG_ROWS = E_PAD // DCH
DEG_RPW = DEG_ROWS // (NC * NS)
N_ACC = 10240
ZROWS = N_ACC // NS
NBUF = 4
IDXBLK = 32
N_IDXBLK = SEG_RPW // IDXBLK
SUP_PER_BLK = IDXBLK // NBUF

_mesh = plsc.VectorSubcoreMesh(core_axis_name="c", subcore_axis_name="s")


def _zero_vmem_2d(ref, nrows):
    zv = jnp.zeros((16,), jnp.float32)

    def body(i, _):
        r = i // 8
        col = (i % 8) * 16
        ref[r, pl.ds(col, 16)] = zv
        return 0

    lax.fori_loop(0, nrows * 8, body, 0)


@functools.partial(
    pl.kernel,
    out_type=jax.ShapeDtypeStruct((2, N_ACC), jnp.float32),
    mesh=_mesh,
    scratch_types=[
        pltpu.VMEM_SHARED((N_ACC,), jnp.float32),
        pltpu.VMEM((640,), jnp.float32),
        pltpu.VMEM((DCH,), jnp.float32),
        pltpu.VMEM((DEG_RPW, DCH), jnp.int32),
        pltpu.SemaphoreType.DMA,
    ],
)
def _deg_kernel(dst2_hbm, out_hbm, acc, zbuf, ones_v, idx_sub, sem):
    c = lax.axis_index("c")
    s = lax.axis_index("s")

    def fill(i, _):
        zbuf[pl.ds(i * 16, 16)] = jnp.zeros((16,), jnp.float32)
        return 0

    lax.fori_loop(0, 40, fill, 0)

    def fill1(i, _):
        ones_v[pl.ds(i * 16, 16)] = jnp.ones((16,), jnp.float32)
        return 0

    lax.fori_loop(0, DCH // 16, fill1, 0)
    base_row = (c * NS + s) * DEG_RPW
    pltpu.sync_copy(dst2_hbm.at[pl.ds(base_row, DEG_RPW)], idx_sub)
    pltpu.sync_copy(zbuf, acc.at[pl.ds(s * ZROWS, ZROWS)])
    plsc.subcore_barrier()

    def fire(g, _):
        pltpu.async_copy(ones_v, acc.at[idx_sub.at[g]], sem, add=True)
        return 0

    lax.fori_loop(0, DEG_RPW, fire, 0)

    def drain(g, _):
        pltpu.make_async_copy(ones_v, acc.at[idx_sub.at[g]], sem).wait()
        return 0

    lax.fori_loop(0, DEG_RPW, drain, 0)
    plsc.subcore_barrier()

    pltpu.sync_copy(acc.at[pl.ds(s * ZROWS, ZROWS)], zbuf)
    pltpu.sync_copy(zbuf, out_hbm.at[c, pl.ds(s * ZROWS, ZROWS)])


@functools.partial(
    pl.kernel,
    out_type=(
        jax.ShapeDtypeStruct((N_ACC, D), jnp.float32),
        jax.ShapeDtypeStruct((N_ACC, D), jnp.float32),
    ),
    mesh=_mesh,
    scratch_types=[
        pltpu.VMEM_SHARED((N_ACC, D), jnp.float32),
        pltpu.VMEM((2, IDXBLK, CHUNK), jnp.int32),
        pltpu.VMEM((2, IDXBLK, CHUNK), jnp.int32),
        pltpu.VMEM((NBUF, CHUNK, D), jnp.float32),
        pltpu.SemaphoreType.DMA((NBUF,)),
        pltpu.SemaphoreType.DMA((NBUF,)),
        pltpu.SemaphoreType.DMA,
    ],
)
def _segsum_kernel(h_hbm, hs_hbm, src2_hbm, dst2_hbm, s_out, t_out,
                   acc, src_sub, dst_sub, rows_v, gsem, ssem, isem):
    c = lax.axis_index("c")
    s = lax.axis_index("s")

    _zero_vmem_2d(rows_v.at[0], CHUNK)
    nz = ZROWS // CHUNK
    for k in range(nz):
        pltpu.async_copy(rows_v.at[0],
                         acc.at[pl.ds(s * ZROWS + k * CHUNK, CHUNK)],
                         gsem.at[0])
    base0 = s * SEG_RPW
    pltpu.sync_copy(src2_hbm.at[pl.ds(base0, IDXBLK)], src_sub.at[0])
    pltpu.sync_copy(dst2_hbm.at[pl.ds(base0, IDXBLK)], dst_sub.at[0])
    for k in range(nz):
        pltpu.make_async_copy(rows_v.at[0],
                              acc.at[pl.ds(s * ZROWS + k * CHUNK, CHUNK)],
                              gsem.at[0]).wait()
    plsc.subcore_barrier()

    def fire_gather(par, b, row):
        @pl.when(c == 0)
        def _():
            pltpu.async_copy(h_hbm.at[src_sub.at[par, row]], rows_v.at[b],
                             gsem.at[b])

        @pl.when(c == 1)
        def _():
            pltpu.async_copy(hs_hbm.at[src_sub.at[par, row]], rows_v.at[b],
                             gsem.at[b])

    def wait_gather(par, b, row):
        pltpu.make_async_copy(h_hbm.at[src_sub.at[par, row]], rows_v.at[b],
                              gsem.at[b]).wait()

    def fire_scatter(par, b, row):
        pltpu.async_copy(rows_v.at[b], acc.at[dst_sub.at[par, row]],
                         ssem.at[b], add=True)

    def wait_scatter(par, b, row):
        pltpu.make_async_copy(rows_v.at[b], acc.at[dst_sub.at[par, row]],
                              ssem.at[b]).wait()

    def block_body(blk, _):
        par = blk % 2
        nxt = (blk + 1) % 2
        more = blk < N_IDXBLK - 1

        @pl.when(more)
        def _():
            base = s * SEG_RPW + (blk + 1) * IDXBLK
            pltpu.async_copy(src2_hbm.at[pl.ds(base, IDXBLK)],
                             src_sub.at[nxt], isem)
            pltpu.async_copy(dst2_hbm.at[pl.ds(base, IDXBLK)],
                             dst_sub.at[nxt], isem)

        for b in range(NBUF):
            fire_gather(par, b, b)

        def super_body(g, _):
            for b in range(NBUF):
                wait_gather(par, b, g * NBUF + b)
                fire_scatter(par, b, g * NBUF + b)

            @pl.when(g < SUP_PER_BLK - 1)
            def _():
                for b in range(NBUF):
                    wait_scatter(par, b, g * NBUF + b)
                    fire_gather(par, b, (g + 1) * NBUF + b)

            return 0

        lax.fori_loop(0, SUP_PER_BLK, super_body, 0)
        for b in range(NBUF):
            wait_scatter(par, b, (SUP_PER_BLK - 1) * NBUF + b)

        @pl.when(more)
        def _():
            base = s * SEG_RPW + (blk + 1) * IDXBLK
            pltpu.make_async_copy(src2_hbm.at[pl.ds(base, IDXBLK)],
                                  src_sub.at[nxt], isem).wait()
            pltpu.make_async_copy(dst2_hbm.at[pl.ds(base, IDXBLK)],
                                  dst_sub.at[nxt], isem).wait()

        return 0

    lax.fori_loop(0, N_IDXBLK, block_body, 0)
    plsc.subcore_barrier()

    sl = pl.ds(s * ZROWS, ZROWS)

    @pl.when(c == 0)
    def _():
        pltpu.sync_copy(acc.at[sl], s_out.at[sl])

    @pl.when(c == 1)
    def _():
        pltpu.sync_copy(acc.at[sl], t_out.at[sl])


BN = 1000
_GRID = N_NODES // BN
_HP = jax.lax.Precision.HIGHEST


def _lin1_body(x_ref, w_ref, b_ref, h_ref):
    h_ref[...] = jnp.dot(x_ref[...], w_ref[...],
                         preferred_element_type=jnp.float32,
                         precision=_HP) + b_ref[...]


def _lin1_call(x, lin1_w, lin1_b):
    return pl.pallas_call(
        _lin1_body,
        grid=(_GRID,),
        in_specs=[
            pl.BlockSpec((BN, D), lambda i: (i, 0)),
            pl.BlockSpec((D, D), lambda i: (0, 0)),
            pl.BlockSpec((1, D), lambda i: (0, 0)),
        ],
        out_specs=pl.BlockSpec((BN, D), lambda i: (i, 0)),
        out_shape=jax.ShapeDtypeStruct((N_NODES, D), jnp.float32),
    )(x, lin1_w, lin1_b.reshape(1, D))


def _prep_body(h_ref, d0_ref, d1_ref, hs_ref, rsd_ref, inv_ref):
    deg = jnp.maximum(d0_ref[...] + d1_ref[...], 1.0)
    rsd = lax.rsqrt(deg)
    hs_ref[...] = h_ref[...] * rsd
    rsd_ref[...] = rsd
    inv_ref[...] = 1.0 / deg


def _prep_call(h, d0, d1):
    return pl.pallas_call(
        _prep_body,
        grid=(_GRID,),
        in_specs=[
            pl.BlockSpec((BN, D), lambda i: (i, 0)),
            pl.BlockSpec((BN, 1), lambda i: (i, 0)),
            pl.BlockSpec((BN, 1), lambda i: (i, 0)),
        ],
        out_specs=[
            pl.BlockSpec((BN, D), lambda i: (i, 0)),
            pl.BlockSpec((BN, 1), lambda i: (i, 0)),
            pl.BlockSpec((BN, 1), lambda i: (i, 0)),
        ],
        out_shape=[
            jax.ShapeDtypeStruct((N_NODES, D), jnp.float32),
            jax.ShapeDtypeStruct((N_NODES, 1), jnp.float32),
            jax.ShapeDtypeStruct((N_NODES, 1), jnp.float32),
        ],
    )(h, d0, d1)


def _elu(v):
    return jnp.where(v > 0, v, jnp.exp(jnp.minimum(v, 0.0)) - 1.0)


def _layer_pre_body(h_ref, sws_ref, sb_ref, ginw_ref, pre_ref):
    h = h_ref[...]
    pre_ref[0] = jnp.dot(h, sws_ref[...], preferred_element_type=jnp.float32,
                         precision=_HP) + sb_ref[...]
    pre_ref[1] = jnp.dot(h, ginw_ref[...], preferred_element_type=jnp.float32,
                         precision=_HP)


def _layer_pre_call(h, sws, sb, ginw):
    full = lambda i: (0, 0)
    blk = lambda i: (i, 0)
    return pl.pallas_call(
        _layer_pre_body,
        grid=(_GRID,),
        in_specs=[
            pl.BlockSpec((BN, D), blk),
            pl.BlockSpec((D, D), full),
            pl.BlockSpec((1, D), full),
            pl.BlockSpec((D, D), full),
        ],
        out_specs=pl.BlockSpec((2, BN, D), lambda i: (0, i, 0)),
        out_shape=jax.ShapeDtypeStruct((2, N_NODES, D), jnp.float32),
    )(h, sws, sb.reshape(1, D), ginw)


def _layer_body(h_ref, s_ref, t_ref, rsd_ref, inv_ref, pre_ref, gw_ref,
                gb_ref, swn_ref, ginw_ref, ginb_ref, naw_ref,
                hn_ref, hns_ref):
    h = h_ref[...]
    S = s_ref[...]
    T = t_ref[...]
    rsd = rsd_ref[...]
    inv = inv_ref[...]
    gcn = jnp.dot(T * rsd, gw_ref[...], preferred_element_type=jnp.float32,
                  precision=_HP) + gb_ref[...]
    sage = (pre_ref[0]
            + jnp.dot(S * inv, swn_ref[...], preferred_element_type=jnp.float32,
                      precision=_HP))
    gin = (pre_ref[1]
           + jnp.dot(S, ginw_ref[...], preferred_element_type=jnp.float32,
                     precision=_HP) + ginb_ref[...])
    w0 = naw_ref[0]
    w1 = naw_ref[1]
    w2 = naw_ref[2]
    o = w0 * _elu(gcn) + w1 * _elu(sage) + w2 * _elu(gin)
    hn_ref[...] = o
    hns_ref[...] = o * rsd


def _layer_call(h, S, T, rsd, inv, pre, gw, gb, swn, ginw, ginb, naw):
    full = lambda i: (0, 0)
    blk = lambda i: (i, 0)
    return pl.pallas_call(
        _layer_body,
        grid=(_GRID,),
        in_specs=[
            pl.BlockSpec((BN, D), blk),
            pl.BlockSpec((BN, D), blk),
            pl.BlockSpec((BN, D), blk),
            pl.BlockSpec((BN, 1), blk),
            pl.BlockSpec((BN, 1), blk),
            pl.BlockSpec((2, BN, D), lambda i: (0, i, 0)),
            pl.BlockSpec((D, D), full),
            pl.BlockSpec((1, D), full),
            pl.BlockSpec((D, D), full),
            pl.BlockSpec((D, D), full),
            pl.BlockSpec((1, D), full),
            pl.BlockSpec(memory_space=pltpu.SMEM),
        ],
        out_specs=[
            pl.BlockSpec((BN, D), blk),
            pl.BlockSpec((BN, D), blk),
        ],
        out_shape=[
            jax.ShapeDtypeStruct((N_NODES, D), jnp.float32),
            jax.ShapeDtypeStruct((N_NODES, D), jnp.float32),
        ],
    )(h, S, T, rsd, inv, pre, gw, gb.reshape(1, D), swn,
      ginw, ginb.reshape(1, D), naw)


def _final_pre_body(x1_ref, x2_ref, lcw_ref, lcb_ref, sc_ref, pre_ref):
    c1 = sc_ref[0]
    c2 = sc_ref[1]
    b = c1 * x1_ref[...]
    c = c2 * x2_ref[...]
    lcw = lcw_ref[...]
    pre_ref[0] = (jnp.dot(b, lcw[D:2 * D], preferred_element_type=jnp.float32,
                          precision=_HP)
                  + jnp.dot(c, lcw[2 * D:3 * D],
                            preferred_element_type=jnp.float32,
                            precision=_HP) + lcb_ref[...])
    pre_ref[1] = jnp.maximum(b, c)
    pre_ref[2] = b + c


def _final_pre_call(x1, x2, lc_w, lc_b, sc_c):
    full = lambda i: (0, 0)
    blk = lambda i: (i, 0)
    return pl.pallas_call(
        _final_pre_body,
        grid=(_GRID,),
        in_specs=[
            pl.BlockSpec((BN, D), blk),
            pl.BlockSpec((BN, D), blk),
            pl.BlockSpec((3 * D, D), full),
            pl.BlockSpec((1, D), full),
            pl.BlockSpec(memory_space=pltpu.SMEM),
        ],
        out_specs=pl.BlockSpec((3, BN, D), lambda i: (0, i, 0)),
        out_shape=jax.ShapeDtypeStruct((3, N_NODES, D), jnp.float32),
    )(x1, x2, lc_w, lc_b.reshape(1, D), sc_c)


def _final_body(x3_ref, pre_ref, lcw_ref, ncw_ref, ncb_ref, law_ref, out_ref):
    a = x3_ref[...]
    l_max = jnp.maximum(a, pre_ref[1])
    l_mean = (a + pre_ref[2]) * (1.0 / 3.0)
    l_cat = (jnp.dot(a, lcw_ref[...], preferred_element_type=jnp.float32,
                     precision=_HP) + pre_ref[0])
    la0 = law_ref[0]
    la1 = law_ref[1]
    la2 = law_ref[2]
    relu = lambda v: jnp.maximum(v, 0.0)
    x5 = la0 * relu(l_max) + la1 * relu(l_cat) + la2 * relu(l_mean)
    out_ref[...] = jnp.dot(x5, ncw_ref[...], preferred_element_type=jnp.float32,
                           precision=_HP) + ncb_ref[...]


def _final_call(x3, pre, lc_w, nc_w, nc_b, la_w):
    full = lambda i: (0, 0)
    blk = lambda i: (i, 0)
    return pl.pallas_call(
        _final_body,
        grid=(_GRID,),
        in_specs=[
            pl.BlockSpec((BN, D), blk),
            pl.BlockSpec((3, BN, D), lambda i: (0, i, 0)),
            pl.BlockSpec((D, D), full),
            pl.BlockSpec((D, C), full),
            pl.BlockSpec((1, C), full),
            pl.BlockSpec(memory_space=pltpu.SMEM),
        ],
        out_specs=pl.BlockSpec((BN, C), blk),
        out_shape=jax.ShapeDtypeStruct((N_NODES, C), jnp.float32),
    )(x3, pre, lc_w[0:D], nc_w, nc_b.reshape(1, C), la_w)


def kernel(x, edge_index, lin1_w, lin1_b, gcn_w, gcn_b, sage_ws, sage_wn,
           sage_b, gin_w, gin_b, lc_w, lc_b, nc_w, nc_b, na_alphas, sc_alphas,
           la_alphas):
    na_w = jax.nn.softmax(na_alphas, axis=-1)
    sc_w = jax.nn.softmax(sc_alphas, axis=-1)
    la_w = jax.nn.softmax(la_alphas, axis=-1)

    src = edge_index[0]
    dst = edge_index[1]
    pad = E_PAD - N_EDGES
    iota = lax.iota(jnp.int32, pad)
    src_p = jnp.concatenate([src, iota % N_NODES]).reshape(N_ROWS, CHUNK)
    dst_flat = jnp.concatenate([dst, N_NODES + (iota % 8)])
    dst_p = dst_flat.reshape(N_ROWS, CHUNK)
    dst_p128 = dst_flat.reshape(DEG_ROWS, DCH)

    deg_pair = _deg_kernel(dst_p128)
    h = _lin1_call(x, lin1_w, lin1_b)
    d0 = deg_pair[0, :N_NODES].reshape(N_NODES, 1)
    d1 = deg_pair[1, :N_NODES].reshape(N_NODES, 1)
    hs, rsd, inv = _prep_call(h, d0, d1)

    S, T = _segsum_kernel(h, hs, src_p, dst_p)
    pre1 = _layer_pre_call(h, sage_ws[0], sage_b[0], gin_w[0])
    x1, x1s = _layer_call(h, S, T, rsd, inv, pre1, gcn_w[0], gcn_b[0],
                          sage_wn[0], gin_w[0], gin_b[0], na_w[0])
    S, T = _segsum_kernel(x1, x1s, src_p, dst_p)
    pre2 = _layer_pre_call(x1, sage_ws[1], sage_b[1], gin_w[1])
    x2, x2s = _layer_call(x1, S, T, rsd, inv, pre2, gcn_w[1], gcn_b[1],
                          sage_wn[1], gin_w[1], gin_b[1], na_w[1])
    sc_c = jnp.stack([sc_w[0, 1], sc_w[1, 1]])
    S, T = _segsum_kernel(x2, x2s, src_p, dst_p)
    pre3 = _layer_pre_call(x2, sage_ws[2], sage_b[2], gin_w[2])
    fpre = _final_pre_call(x1, x2, lc_w, lc_b, sc_c)
    x3, _ = _layer_call(x2, S, T, rsd, inv, pre3, gcn_w[2], gcn_b[2],
                        sage_wn[2], gin_w[2], gin_b[2], na_w[2])

    return _final_call(x3, fpre, lc_w, nc_w, nc_b, la_w[0])
